# baseline (speedup 1.0000x reference)
import functools

import jax
import jax.numpy as jnp
from jax import lax
from jax.experimental import pallas as pl
from jax.experimental.pallas import tpu as pltpu
from jax.experimental.pallas import tpu_sc as plsc

_N = 10000
_E = 320000
_D = 128
_H = _D // 2

_NW = 32
_EPT = _E // _NW
_S = 400
_SUB = 80
_K = _S // _SUB
_NO = _EPT // _S
_NP = 10240
_STRIPE = _NP // 16
_C2 = 128
_NC2 = _EPT // _C2
_TAIL = _EPT - _NC2 * _C2


def _proj_body(n_ref, w_ref, ps_ref, pr_ref, pn_ref):
    o = jnp.dot(n_ref[...], w_ref[...], preferred_element_type=jnp.float32)
    ps_ref[...] = o[:, 0:_D]
    pr_ref[...] = o[:, _D:2 * _D]
    pn_ref[...] = o[:, 2 * _D:3 * _D]


def _proj(nodes, wcat):
    nb = 2000
    blk = lambda i: (i, 0)
    return pl.pallas_call(
        _proj_body,
        grid=(_N // nb,),
        in_specs=[pl.BlockSpec((nb, _D), blk),
                  pl.BlockSpec((_D, 3 * _D), lambda i: (0, 0))],
        out_specs=[pl.BlockSpec((nb, _D), blk),
                   pl.BlockSpec((nb, _D), blk),
                   pl.BlockSpec((nb, _D), blk)],
        out_shape=[jax.ShapeDtypeStruct((_N, _D), jnp.float32),
                   jax.ShapeDtypeStruct((_N, _D), jnp.float32),
                   jax.ShapeDtypeStruct((_N, _D), jnp.float32)],
    )(nodes, wcat)


def _gather_body(ps_hbm, pr_hbm, snd_hbm, rcv_hbm, os_hbm, or_hbm,
                 idx0, idx1, rows,
                 sem_i0, sem_i1, sem_g0, sem_g1, sem_w0, sem_w1):
    wid = lax.axis_index("s") * 2 + lax.axis_index("c")
    base0 = wid * _EPT
    idx_v = (idx0, idx1)
    sem_i = (sem_i0, sem_i1)
    sem_g = (sem_g0, sem_g1)
    sem_w = (sem_w0, sem_w1)

    def one_pass(tab_hbm, ix_hbm, out_hbm):
        def issue_idx(g):
            b = g & 1
            base = base0 + g * _S
            return [pltpu.async_copy(ix_hbm.at[pl.ds(base, _S)], idx_v[b],
                                     sem_i[b])]

        def issue_gathers(g):
            b = g & 1
            hs = []
            for k in range(_K):
                sl = pl.ds(k * _SUB, _SUB)
                hs.append(pltpu.async_copy(tab_hbm.at[idx_v[b].at[sl]],
                                           rows.at[b, sl], sem_g[b]))
            return hs

        def issue_writebacks(g):
            b = g & 1
            base = base0 + g * _S
            return [pltpu.async_copy(rows.at[b], out_hbm.at[pl.ds(base, _S)],
                                     sem_w[b])]

        h_i = [None, None]
        h_g = [None, None]
        h_w = [None, None]
        h_i[0] = issue_idx(0)
        for g in range(_NO):
            b = g & 1
            if h_w[b] is not None:
                for h in h_w[b]:
                    h.wait()
            for h in h_i[b]:
                h.wait()
            h_g[b] = issue_gathers(g)
            if g >= 1:
                for h in h_g[1 - b]:
                    h.wait()
                h_w[1 - b] = issue_writebacks(g - 1)
            if g + 1 < _NO:
                h_i[1 - b] = issue_idx(g + 1)
        bl = (_NO - 1) & 1
        for h in h_g[bl]:
            h.wait()
        h_w[bl] = issue_writebacks(_NO - 1)
        for hw in h_w:
            if hw is not None:
                for h in hw:
                    h.wait()

    one_pass(ps_hbm, snd_hbm, os_hbm)
    one_pass(pr_hbm, rcv_hbm, or_hbm)


def _gather(ps, pr, senders, receivers):
    mesh = plsc.VectorSubcoreMesh(core_axis_name="c", subcore_axis_name="s")
    f = functools.partial(
        pl.kernel,
        out_type=[jax.ShapeDtypeStruct((_E, _D), jnp.float32),
                  jax.ShapeDtypeStruct((_E, _D), jnp.float32)],
        mesh=mesh,
        scratch_types=[pltpu.VMEM((_S,), jnp.int32),
                       pltpu.VMEM((_S,), jnp.int32),
                       pltpu.VMEM((2, _S, _D), jnp.float32),
                       pltpu.SemaphoreType.DMA,
                       pltpu.SemaphoreType.DMA,
                       pltpu.SemaphoreType.DMA,
                       pltpu.SemaphoreType.DMA,
                       pltpu.SemaphoreType.DMA,
                       pltpu.SemaphoreType.DMA],
    )(_gather_body)
    return f(ps, pr, senders, receivers)


def _edge_body(e_ref, ps_ref, pr_ref, we_ref, b1_ref, w2_ref, b2_ref, o_ref):
    x = jnp.dot(e_ref[...].astype(jnp.bfloat16),
                we_ref[...].astype(jnp.bfloat16),
                preferred_element_type=jnp.float32)
    h = jnp.maximum(x + ps_ref[...] + pr_ref[...] + b1_ref[...], 0.0)
    o_ref[...] = (jnp.dot(h.astype(jnp.bfloat16),
                          w2_ref[...].astype(jnp.bfloat16),
                          preferred_element_type=jnp.float32)
                  + b2_ref[...])


def _edge_mlp(edges, psg, prg, we, be1, we2, be2):
    eb = 1280
    blk = lambda i: (i, 0)
    wspec = pl.BlockSpec((_D, _D), lambda i: (0, 0))
    bspec = pl.BlockSpec((1, _D), lambda i: (0, 0))
    return pl.pallas_call(
        _edge_body,
        grid=(_E // eb,),
        in_specs=[pl.BlockSpec((eb, _D), blk),
                  pl.BlockSpec((eb, _D), blk),
                  pl.BlockSpec((eb, _D), blk),
                  wspec, bspec, wspec, bspec],
        out_specs=pl.BlockSpec((eb, _D), blk),
        out_shape=jax.ShapeDtypeStruct((_E, _D), jnp.float32),
    )(edges, psg, prg, we, be1.reshape(1, _D), we2, be2.reshape(1, _D))


def _scatter_body(ne_hbm, rcv_hbm, zeros_hbm, sums_hbm,
                  idx0, idx1, idxt, rows, acc_s, sem_l0, sem_l1):
    cid = lax.axis_index("c")
    sid = lax.axis_index("s")
    wid = sid * 2 + cid
    srow = sid * _STRIPE
    idx_v = (idx0, idx1)
    sem_l = (sem_l0, sem_l1)
    pltpu.sync_copy(zeros_hbm.at[pl.ds(srow, _STRIPE)],
                    acc_s.at[pl.ds(srow, _STRIPE)])
    plsc.subcore_barrier()

    base0 = wid * _EPT

    def issue_loads(g):
        b = g & 1
        base = base0 + g * _C2
        return [pltpu.async_copy(ne_hbm.at[pl.ds(base, _C2)], rows.at[b],
                                 sem_l[b]),
                pltpu.async_copy(rcv_hbm.at[pl.ds(base, _C2)], idx_v[b],
                                 sem_l[b])]

    h_l = [None, None]
    h_l[0] = issue_loads(0)
    for g in range(_NC2):
        b = g & 1
        for h in h_l[b]:
            h.wait()
        if g + 1 < _NC2:
            h_l[1 - b] = issue_loads(g + 1)
        pltpu.sync_copy(rows.at[b], acc_s.at[idx_v[b]], add=True)

    tbase = base0 + _NC2 * _C2
    pltpu.sync_copy(rcv_hbm.at[pl.ds(tbase, _TAIL)], idxt)
    pltpu.sync_copy(ne_hbm.at[pl.ds(tbase, _TAIL)], rows.at[0, pl.ds(0, _TAIL)])
    pltpu.sync_copy(rows.at[0, pl.ds(0, _TAIL)], acc_s.at[idxt], add=True)

    plsc.subcore_barrier()
    pltpu.sync_copy(acc_s.at[pl.ds(srow, _STRIPE)],
                    sums_hbm.at[cid, pl.ds(srow, _STRIPE)])


def _scatter(new_edges, receivers, zeros):
    mesh = plsc.VectorSubcoreMesh(core_axis_name="c", subcore_axis_name="s")
    f = functools.partial(
        pl.kernel,
        out_type=jax.ShapeDtypeStruct((2, _NP, _D), jnp.float32),
        mesh=mesh,
        scratch_types=[pltpu.VMEM((_C2,), jnp.int32),
                       pltpu.VMEM((_C2,), jnp.int32),
                       pltpu.VMEM((_TAIL,), jnp.int32),
                       pltpu.VMEM((2, _C2, _D), jnp.float32),
                       pltpu.VMEM_SHARED((_NP, _D), jnp.float32),
                       pltpu.SemaphoreType.DMA,
                       pltpu.SemaphoreType.DMA],
    )(_scatter_body)
    return f(new_edges, receivers, zeros)


def _counts_body(rcv_hbm, zeros_hbm, ones_hbm, cnts_hbm,
                 idx0, idx1, idxt, ones_v, acc_c, sem_i0, sem_i1):
    cid = lax.axis_index("c")
    sid = lax.axis_index("s")
    wid = sid * 2 + cid
    srow = sid * _STRIPE
    idx_v = (idx0, idx1)
    sem_i = (sem_i0, sem_i1)
    pltpu.sync_copy(zeros_hbm.at[pl.ds(srow, _STRIPE)],
                    acc_c.at[pl.ds(srow, _STRIPE)])
    pltpu.sync_copy(ones_hbm, ones_v)
    plsc.subcore_barrier()

    base0 = wid * _EPT

    def issue_idx(g):
        b = g & 1
        base = base0 + g * _C2
        return [pltpu.async_copy(rcv_hbm.at[pl.ds(base, _C2)], idx_v[b],
                                 sem_i[b])]

    h_i = [None, None]
    h_i[0] = issue_idx(0)
    for g in range(_NC2):
        b = g & 1
        for h in h_i[b]:
            h.wait()
        if g + 1 < _NC2:
            h_i[1 - b] = issue_idx(g + 1)
        pltpu.sync_copy(ones_v, acc_c.at[idx_v[b]], add=True)

    tbase = base0 + _NC2 * _C2
    pltpu.sync_copy(rcv_hbm.at[pl.ds(tbase, _TAIL)], idxt)
    pltpu.sync_copy(ones_v.at[pl.ds(0, _TAIL)], acc_c.at[idxt], add=True)

    plsc.subcore_barrier()
    pltpu.sync_copy(acc_c.at[pl.ds(srow, _STRIPE)],
                    cnts_hbm.at[cid, pl.ds(srow, _STRIPE)])


def _counts(receivers, zeros, ones):
    mesh = plsc.VectorSubcoreMesh(core_axis_name="c", subcore_axis_name="s")
    f = functools.partial(
        pl.kernel,
        out_type=jax.ShapeDtypeStruct((2, _NP, _D), jnp.float32),
        mesh=mesh,
        scratch_types=[pltpu.VMEM((_C2,), jnp.int32),
                       pltpu.VMEM((_C2,), jnp.int32),
                       pltpu.VMEM((_TAIL,), jnp.int32),
                       pltpu.VMEM((_C2, _D), jnp.float32),
                       pltpu.VMEM_SHARED((_NP, _D), jnp.float32),
                       pltpu.SemaphoreType.DMA,
                       pltpu.SemaphoreType.DMA],
    )(_counts_body)
    return f(receivers, zeros, ones)


def _node_body(s0_ref, s1_ref, c0_ref, c1_ref, pn_ref, w_ref, b1_ref,
               w2_ref, b2_ref, o_ref):
    sums = s0_ref[...] + s1_ref[...]
    cnt = jnp.max(c0_ref[...] + c1_ref[...], axis=1, keepdims=True)
    agg = sums / jnp.maximum(cnt, 1.0)
    hn = jnp.maximum(
        pn_ref[...]
        + jnp.dot(agg, w_ref[...], preferred_element_type=jnp.float32)
        + b1_ref[...], 0.0)
    o_ref[...] = (jnp.dot(hn, w2_ref[...], preferred_element_type=jnp.float32)
                  + b2_ref[...])


def _node_mlp(s0, s1, c0, c1, pn, wn1b, bn1, wn2, bn2):
    nb = 1000
    blk = lambda i: (i, 0)
    wspec = pl.BlockSpec((_D, _D), lambda i: (0, 0))
    bspec = pl.BlockSpec((1, _D), lambda i: (0, 0))
    return pl.pallas_call(
        _node_body,
        grid=(_N // nb,),
        in_specs=[pl.BlockSpec((nb, _D), blk),
                  pl.BlockSpec((nb, _D), blk),
                  pl.BlockSpec((nb, _D), blk),
                  pl.BlockSpec((nb, _D), blk),
                  pl.BlockSpec((nb, _D), blk),
                  wspec, bspec, wspec, bspec],
        out_specs=pl.BlockSpec((nb, _D), blk),
        out_shape=jax.ShapeDtypeStruct((_N, _D), jnp.float32),
    )(s0, s1, c0, c1, pn, wn1b, bn1.reshape(1, _D), wn2, bn2.reshape(1, _D))


def kernel(nodes, edges, senders, receivers,
           We1, be1, We2, be2, Wn1, bn1, Wn2, bn2):
    we = We1[0:_D]
    ws = We1[_D:2 * _D]
    wr = We1[2 * _D:3 * _D]
    wn1a = Wn1[0:_D]
    wn1b = Wn1[_D:2 * _D]

    ps, pr, pn = _proj(nodes, jnp.concatenate([ws, wr, wn1a], axis=1))

    psg, prg = _gather(ps, pr, senders, receivers)
    new_edges = _edge_mlp(edges, psg, prg, we, be1, We2, be2)

    zeros = jnp.zeros((_NP, _D), jnp.float32)
    ones = jnp.ones((_C2, _D), jnp.float32)
    cnts = _counts(receivers, zeros, ones)
    sums = _scatter(new_edges, receivers, zeros)

    new_nodes = _node_mlp(sums[0, :_N], sums[1, :_N], cnts[0, :_N],
                          cnts[1, :_N], pn, wn1b, bn1, Wn2, bn2)
    return (new_nodes, new_edges)


# speedup vs baseline: 1.1067x; 1.1067x over previous
import functools

import jax
import jax.numpy as jnp
from jax import lax
from jax.experimental import pallas as pl
from jax.experimental.pallas import tpu as pltpu
from jax.experimental.pallas import tpu_sc as plsc

_N = 10000
_E = 320000
_D = 128
_H = _D // 2

_NW = 32
_EPT = _E // _NW
_S = 400
_SUB = 80
_K = _S // _SUB
_NO = _EPT // _S
_NP = 10240
_STRIPE = _NP // 16
_C2 = 128
_NC2 = _EPT // _C2
_TAIL = _EPT - _NC2 * _C2


def _proj_body(n_ref, w_ref, ps_ref, pr_ref, pn_ref):
    o = jnp.dot(n_ref[...], w_ref[...], preferred_element_type=jnp.float32)
    ps_ref[...] = o[:, 0:_D]
    pr_ref[...] = o[:, _D:2 * _D]
    pn_ref[...] = o[:, 2 * _D:3 * _D]


def _proj(nodes, wcat):
    nb = 2000
    blk = lambda i: (i, 0)
    return pl.pallas_call(
        _proj_body,
        grid=(_N // nb,),
        in_specs=[pl.BlockSpec((nb, _D), blk),
                  pl.BlockSpec((_D, 3 * _D), lambda i: (0, 0))],
        out_specs=[pl.BlockSpec((nb, _D), blk),
                   pl.BlockSpec((nb, _D), blk),
                   pl.BlockSpec((nb, _D), blk)],
        out_shape=[jax.ShapeDtypeStruct((_N, _D), jnp.float32),
                   jax.ShapeDtypeStruct((_N, _D), jnp.float32),
                   jax.ShapeDtypeStruct((_N, _D), jnp.float32)],
    )(nodes, wcat)


_GS = 120
_GNO = _EPT // _GS
_GT = _EPT - _GNO * _GS


def _gather_body(ps_hbm, pr_hbm, snd_hbm, rcv_hbm, o_hbm,
                 si0, si1, ri0, ri1, pbuf, rbuf,
                 sem_i0, sem_i1, sem_g0, sem_g1, sem_w0, sem_w1):
    wid = lax.axis_index("s") * 2 + lax.axis_index("c")
    base0 = wid * _EPT
    si = (si0, si1)
    ri = (ri0, ri1)
    sem_i = (sem_i0, sem_i1)
    sem_g = (sem_g0, sem_g1)
    sem_w = (sem_w0, sem_w1)

    def issue_idx(g):
        b = g & 1
        base = base0 + g * _GS
        return [pltpu.async_copy(snd_hbm.at[pl.ds(base, _GS)], si[b],
                                 sem_i[b]),
                pltpu.async_copy(rcv_hbm.at[pl.ds(base, _GS)], ri[b],
                                 sem_i[b])]

    def issue_gathers(g):
        b = g & 1
        return [pltpu.async_copy(ps_hbm.at[si[b]], pbuf.at[b], sem_g[b]),
                pltpu.async_copy(pr_hbm.at[ri[b]], rbuf.at[b], sem_g[b])]

    def add_rows(b):
        def body(i, carry):
            for j in range(_D // 16):
                sl = pl.ds(j * 16, 16)
                pbuf[b, i, sl] = pbuf[b, i, sl] + rbuf[b, i, sl]
            return carry
        lax.fori_loop(0, _GS, body, 0)

    def issue_writeback(g):
        b = g & 1
        base = base0 + g * _GS
        return [pltpu.async_copy(pbuf.at[b], o_hbm.at[pl.ds(base, _GS)],
                                 sem_w[b])]

    h_i = [None, None]
    h_g = [None, None]
    h_w = [None, None]
    h_i[0] = issue_idx(0)
    for g in range(_GNO):
        b = g & 1
        if h_w[b] is not None:
            for h in h_w[b]:
                h.wait()
        for h in h_i[b]:
            h.wait()
        h_g[b] = issue_gathers(g)
        if g >= 1:
            for h in h_g[1 - b]:
                h.wait()
            add_rows(1 - b)
            h_w[1 - b] = issue_writeback(g - 1)
            if g + 1 < _GNO:
                h_i[1 - b] = issue_idx(g + 1)
        else:
            h_i[1] = issue_idx(1)
    bl = (_GNO - 1) & 1
    for h in h_g[bl]:
        h.wait()
    add_rows(bl)
    h_w[bl] = issue_writeback(_GNO - 1)
    for hw in h_w:
        if hw is not None:
            for h in hw:
                h.wait()

    tbase = base0 + _GNO * _GS
    tsl = pl.ds(0, _GT)
    pltpu.sync_copy(snd_hbm.at[pl.ds(tbase, _GT)], si0.at[tsl])
    pltpu.sync_copy(rcv_hbm.at[pl.ds(tbase, _GT)], ri0.at[tsl])
    pltpu.async_copy(ps_hbm.at[si0.at[tsl]], pbuf.at[0, tsl], sem_g0).wait()
    pltpu.async_copy(pr_hbm.at[ri0.at[tsl]], rbuf.at[0, tsl], sem_g0).wait()

    def tbody(i, carry):
        for j in range(_D // 16):
            sl = pl.ds(j * 16, 16)
            pbuf[0, i, sl] = pbuf[0, i, sl] + rbuf[0, i, sl]
        return carry
    lax.fori_loop(0, _GT, tbody, 0)
    pltpu.sync_copy(pbuf.at[0, tsl], o_hbm.at[pl.ds(tbase, _GT)])


def _gather(ps, pr, senders, receivers):
    mesh = plsc.VectorSubcoreMesh(core_axis_name="c", subcore_axis_name="s")
    f = functools.partial(
        pl.kernel,
        out_type=jax.ShapeDtypeStruct((_E, _D), jnp.float32),
        mesh=mesh,
        scratch_types=[pltpu.VMEM((_GS,), jnp.int32),
                       pltpu.VMEM((_GS,), jnp.int32),
                       pltpu.VMEM((_GS,), jnp.int32),
                       pltpu.VMEM((_GS,), jnp.int32),
                       pltpu.VMEM((2, _GS, _D), jnp.float32),
                       pltpu.VMEM((2, _GS, _D), jnp.float32),
                       pltpu.SemaphoreType.DMA,
                       pltpu.SemaphoreType.DMA,
                       pltpu.SemaphoreType.DMA,
                       pltpu.SemaphoreType.DMA,
                       pltpu.SemaphoreType.DMA,
                       pltpu.SemaphoreType.DMA],
    )(_gather_body)
    return f(ps, pr, senders, receivers)


def _edge_body(e_ref, g_ref, we_ref, b1_ref, w2_ref, b2_ref, o_ref):
    x = jnp.dot(e_ref[...], we_ref[...], preferred_element_type=jnp.float32)
    h = jnp.maximum(x + g_ref[...] + b1_ref[...], 0.0)
    o_ref[...] = (jnp.dot(h, w2_ref[...], preferred_element_type=jnp.float32)
                  + b2_ref[...])


def _edge_mlp(edges, gsum, we, be1, we2, be2):
    eb = 1280
    blk = lambda i: (i, 0)
    wspec = pl.BlockSpec((_D, _D), lambda i: (0, 0))
    bspec = pl.BlockSpec((1, _D), lambda i: (0, 0))
    return pl.pallas_call(
        _edge_body,
        grid=(_E // eb,),
        in_specs=[pl.BlockSpec((eb, _D), blk),
                  pl.BlockSpec((eb, _D), blk),
                  wspec, bspec, wspec, bspec],
        out_specs=pl.BlockSpec((eb, _D), blk),
        out_shape=jax.ShapeDtypeStruct((_E, _D), jnp.float32),
    )(edges, gsum, we, be1.reshape(1, _D), we2, be2.reshape(1, _D))


def _scatter_body(ne_hbm, rcv_hbm, zeros_hbm, sums_hbm,
                  idx0, idx1, idxt, rows, acc_s, sem_l0, sem_l1):
    cid = lax.axis_index("c")
    sid = lax.axis_index("s")
    wid = sid * 2 + cid
    srow = sid * _STRIPE
    idx_v = (idx0, idx1)
    sem_l = (sem_l0, sem_l1)
    pltpu.sync_copy(zeros_hbm.at[pl.ds(srow, _STRIPE)],
                    acc_s.at[pl.ds(srow, _STRIPE)])
    plsc.subcore_barrier()

    base0 = wid * _EPT

    def issue_loads(g):
        b = g & 1
        base = base0 + g * _C2
        return [pltpu.async_copy(ne_hbm.at[pl.ds(base, _C2)], rows.at[b],
                                 sem_l[b]),
                pltpu.async_copy(rcv_hbm.at[pl.ds(base, _C2)], idx_v[b],
                                 sem_l[b])]

    h_l = [None, None]
    h_l[0] = issue_loads(0)
    for g in range(_NC2):
        b = g & 1
        for h in h_l[b]:
            h.wait()
        if g + 1 < _NC2:
            h_l[1 - b] = issue_loads(g + 1)
        pltpu.sync_copy(rows.at[b], acc_s.at[idx_v[b]], add=True)

    tbase = base0 + _NC2 * _C2
    pltpu.sync_copy(rcv_hbm.at[pl.ds(tbase, _TAIL)], idxt)
    pltpu.sync_copy(ne_hbm.at[pl.ds(tbase, _TAIL)], rows.at[0, pl.ds(0, _TAIL)])
    pltpu.sync_copy(rows.at[0, pl.ds(0, _TAIL)], acc_s.at[idxt], add=True)

    plsc.subcore_barrier()
    pltpu.sync_copy(acc_s.at[pl.ds(srow, _STRIPE)],
                    sums_hbm.at[cid, pl.ds(srow, _STRIPE)])


def _scatter(new_edges, receivers, zeros):
    mesh = plsc.VectorSubcoreMesh(core_axis_name="c", subcore_axis_name="s")
    f = functools.partial(
        pl.kernel,
        out_type=jax.ShapeDtypeStruct((2, _NP, _D), jnp.float32),
        mesh=mesh,
        scratch_types=[pltpu.VMEM((_C2,), jnp.int32),
                       pltpu.VMEM((_C2,), jnp.int32),
                       pltpu.VMEM((_TAIL,), jnp.int32),
                       pltpu.VMEM((2, _C2, _D), jnp.float32),
                       pltpu.VMEM_SHARED((_NP, _D), jnp.float32),
                       pltpu.SemaphoreType.DMA,
                       pltpu.SemaphoreType.DMA],
    )(_scatter_body)
    return f(new_edges, receivers, zeros)


def _counts_body(rcv_hbm, zeros_hbm, ones_hbm, cnts_hbm,
                 idx0, idx1, idxt, ones_v, acc_c, sem_i0, sem_i1):
    cid = lax.axis_index("c")
    sid = lax.axis_index("s")
    wid = sid * 2 + cid
    srow = sid * _STRIPE
    idx_v = (idx0, idx1)
    sem_i = (sem_i0, sem_i1)
    pltpu.sync_copy(zeros_hbm.at[pl.ds(srow, _STRIPE)],
                    acc_c.at[pl.ds(srow, _STRIPE)])
    pltpu.sync_copy(ones_hbm, ones_v)
    plsc.subcore_barrier()

    base0 = wid * _EPT

    def issue_idx(g):
        b = g & 1
        base = base0 + g * _C2
        return [pltpu.async_copy(rcv_hbm.at[pl.ds(base, _C2)], idx_v[b],
                                 sem_i[b])]

    h_i = [None, None]
    h_i[0] = issue_idx(0)
    for g in range(_NC2):
        b = g & 1
        for h in h_i[b]:
            h.wait()
        if g + 1 < _NC2:
            h_i[1 - b] = issue_idx(g + 1)
        pltpu.sync_copy(ones_v, acc_c.at[idx_v[b]], add=True)

    tbase = base0 + _NC2 * _C2
    pltpu.sync_copy(rcv_hbm.at[pl.ds(tbase, _TAIL)], idxt)
    pltpu.sync_copy(ones_v.at[pl.ds(0, _TAIL)], acc_c.at[idxt], add=True)

    plsc.subcore_barrier()
    pltpu.sync_copy(acc_c.at[pl.ds(srow, _STRIPE)],
                    cnts_hbm.at[cid, pl.ds(srow, _STRIPE)])


def _counts(receivers, zeros, ones):
    mesh = plsc.VectorSubcoreMesh(core_axis_name="c", subcore_axis_name="s")
    f = functools.partial(
        pl.kernel,
        out_type=jax.ShapeDtypeStruct((2, _NP, _D), jnp.float32),
        mesh=mesh,
        scratch_types=[pltpu.VMEM((_C2,), jnp.int32),
                       pltpu.VMEM((_C2,), jnp.int32),
                       pltpu.VMEM((_TAIL,), jnp.int32),
                       pltpu.VMEM((_C2, _D), jnp.float32),
                       pltpu.VMEM_SHARED((_NP, _D), jnp.float32),
                       pltpu.SemaphoreType.DMA,
                       pltpu.SemaphoreType.DMA],
    )(_counts_body)
    return f(receivers, zeros, ones)


def _node_body(s0_ref, s1_ref, c0_ref, c1_ref, pn_ref, w_ref, b1_ref,
               w2_ref, b2_ref, o_ref):
    sums = s0_ref[...] + s1_ref[...]
    cnt = jnp.max(c0_ref[...] + c1_ref[...], axis=1, keepdims=True)
    agg = sums / jnp.maximum(cnt, 1.0)
    hn = jnp.maximum(
        pn_ref[...]
        + jnp.dot(agg, w_ref[...], preferred_element_type=jnp.float32)
        + b1_ref[...], 0.0)
    o_ref[...] = (jnp.dot(hn, w2_ref[...], preferred_element_type=jnp.float32)
                  + b2_ref[...])


def _node_mlp(s0, s1, c0, c1, pn, wn1b, bn1, wn2, bn2):
    nb = 1000
    blk = lambda i: (i, 0)
    wspec = pl.BlockSpec((_D, _D), lambda i: (0, 0))
    bspec = pl.BlockSpec((1, _D), lambda i: (0, 0))
    return pl.pallas_call(
        _node_body,
        grid=(_N // nb,),
        in_specs=[pl.BlockSpec((nb, _D), blk),
                  pl.BlockSpec((nb, _D), blk),
                  pl.BlockSpec((nb, _D), blk),
                  pl.BlockSpec((nb, _D), blk),
                  pl.BlockSpec((nb, _D), blk),
                  wspec, bspec, wspec, bspec],
        out_specs=pl.BlockSpec((nb, _D), blk),
        out_shape=jax.ShapeDtypeStruct((_N, _D), jnp.float32),
    )(s0, s1, c0, c1, pn, wn1b, bn1.reshape(1, _D), wn2, bn2.reshape(1, _D))


def kernel(nodes, edges, senders, receivers,
           We1, be1, We2, be2, Wn1, bn1, Wn2, bn2):
    we = We1[0:_D]
    ws = We1[_D:2 * _D]
    wr = We1[2 * _D:3 * _D]
    wn1a = Wn1[0:_D]
    wn1b = Wn1[_D:2 * _D]

    ps, pr, pn = _proj(nodes, jnp.concatenate([ws, wr, wn1a], axis=1))

    gsum = _gather(ps, pr, senders, receivers)
    new_edges = _edge_mlp(edges, gsum, we, be1, We2, be2)

    zeros = jnp.zeros((_NP, _D), jnp.float32)
    ones = jnp.ones((_C2, _D), jnp.float32)
    cnts = _counts(receivers, zeros, ones)
    sums = _scatter(new_edges, receivers, zeros)

    new_nodes = _node_mlp(sums[0, :_N], sums[1, :_N], cnts[0, :_N],
                          cnts[1, :_N], pn, wn1b, bn1, Wn2, bn2)
    return (new_nodes, new_edges)


# speedup vs baseline: 1.1075x; 1.0007x over previous
import functools

import jax
import jax.numpy as jnp
from jax import lax
from jax.experimental import pallas as pl
from jax.experimental.pallas import tpu as pltpu
from jax.experimental.pallas import tpu_sc as plsc

_N = 10000
_E = 320000
_D = 128
_H = _D // 2

_NW = 32
_EPT = _E // _NW
_S = 400
_SUB = 80
_K = _S // _SUB
_NO = _EPT // _S
_NP = 10240
_STRIPE = _NP // 16
_C2 = 128
_NC2 = _EPT // _C2
_TAIL = _EPT - _NC2 * _C2


def _proj_body(n_ref, w_ref, ps_ref, pr_ref, pn_ref):
    o = jnp.dot(n_ref[...], w_ref[...], preferred_element_type=jnp.float32)
    ps_ref[...] = o[:, 0:_D]
    pr_ref[...] = o[:, _D:2 * _D]
    pn_ref[...] = o[:, 2 * _D:3 * _D]


def _proj(nodes, wcat):
    nb = 2000
    blk = lambda i: (i, 0)
    return pl.pallas_call(
        _proj_body,
        grid=(_N // nb,),
        in_specs=[pl.BlockSpec((nb, _D), blk),
                  pl.BlockSpec((_D, 3 * _D), lambda i: (0, 0))],
        out_specs=[pl.BlockSpec((nb, _D), blk),
                   pl.BlockSpec((nb, _D), blk),
                   pl.BlockSpec((nb, _D), blk)],
        out_shape=[jax.ShapeDtypeStruct((_N, _D), jnp.float32),
                   jax.ShapeDtypeStruct((_N, _D), jnp.float32),
                   jax.ShapeDtypeStruct((_N, _D), jnp.float32)],
    )(nodes, wcat)


_GS = 120
_GNO = _EPT // _GS
_GT = _EPT - _GNO * _GS


def _gather_body(ps_hbm, pr_hbm, snd_hbm, rcv_hbm, o_hbm,
                 si0, si1, ri0, ri1, pbuf, rbuf,
                 sem_i0, sem_i1, sem_g0, sem_g1, sem_w0, sem_w1):
    wid = lax.axis_index("s") * 2 + lax.axis_index("c")
    base0 = wid * _EPT
    si = (si0, si1)
    ri = (ri0, ri1)
    sem_i = (sem_i0, sem_i1)
    sem_g = (sem_g0, sem_g1)
    sem_w = (sem_w0, sem_w1)

    def issue_idx(g):
        b = g & 1
        base = base0 + g * _GS
        return [pltpu.async_copy(snd_hbm.at[pl.ds(base, _GS)], si[b],
                                 sem_i[b]),
                pltpu.async_copy(rcv_hbm.at[pl.ds(base, _GS)], ri[b],
                                 sem_i[b])]

    def issue_gathers(g):
        b = g & 1
        return [pltpu.async_copy(ps_hbm.at[si[b]], pbuf.at[b], sem_g[b]),
                pltpu.async_copy(pr_hbm.at[ri[b]], rbuf.at[b], sem_g[b])]

    def add_rows(b):
        def body(i, carry):
            for j in range(_D // 16):
                sl = pl.ds(j * 16, 16)
                pbuf[b, i, sl] = pbuf[b, i, sl] + rbuf[b, i, sl]
            return carry
        lax.fori_loop(0, _GS, body, 0)

    def issue_writeback(g):
        b = g & 1
        base = base0 + g * _GS
        return [pltpu.async_copy(pbuf.at[b], o_hbm.at[pl.ds(base, _GS)],
                                 sem_w[b])]

    h_i = [None, None]
    h_g = [None, None]
    h_w = [None, None]
    h_i[0] = issue_idx(0)
    for g in range(_GNO):
        b = g & 1
        if h_w[b] is not None:
            for h in h_w[b]:
                h.wait()
        for h in h_i[b]:
            h.wait()
        h_g[b] = issue_gathers(g)
        if g >= 1:
            for h in h_g[1 - b]:
                h.wait()
            add_rows(1 - b)
            h_w[1 - b] = issue_writeback(g - 1)
            if g + 1 < _GNO:
                h_i[1 - b] = issue_idx(g + 1)
        else:
            h_i[1] = issue_idx(1)
    bl = (_GNO - 1) & 1
    for h in h_g[bl]:
        h.wait()
    add_rows(bl)
    h_w[bl] = issue_writeback(_GNO - 1)
    for hw in h_w:
        if hw is not None:
            for h in hw:
                h.wait()

    tbase = base0 + _GNO * _GS
    tsl = pl.ds(0, _GT)
    pltpu.sync_copy(snd_hbm.at[pl.ds(tbase, _GT)], si0.at[tsl])
    pltpu.sync_copy(rcv_hbm.at[pl.ds(tbase, _GT)], ri0.at[tsl])
    pltpu.async_copy(ps_hbm.at[si0.at[tsl]], pbuf.at[0, tsl], sem_g0).wait()
    pltpu.async_copy(pr_hbm.at[ri0.at[tsl]], rbuf.at[0, tsl], sem_g0).wait()

    def tbody(i, carry):
        for j in range(_D // 16):
            sl = pl.ds(j * 16, 16)
            pbuf[0, i, sl] = pbuf[0, i, sl] + rbuf[0, i, sl]
        return carry
    lax.fori_loop(0, _GT, tbody, 0)
    pltpu.sync_copy(pbuf.at[0, tsl], o_hbm.at[pl.ds(tbase, _GT)])


def _gather(ps, pr, senders, receivers):
    mesh = plsc.VectorSubcoreMesh(core_axis_name="c", subcore_axis_name="s")
    f = functools.partial(
        pl.kernel,
        out_type=jax.ShapeDtypeStruct((_E, _D), jnp.float32),
        mesh=mesh,
        scratch_types=[pltpu.VMEM((_GS,), jnp.int32),
                       pltpu.VMEM((_GS,), jnp.int32),
                       pltpu.VMEM((_GS,), jnp.int32),
                       pltpu.VMEM((_GS,), jnp.int32),
                       pltpu.VMEM((2, _GS, _D), jnp.float32),
                       pltpu.VMEM((2, _GS, _D), jnp.float32),
                       pltpu.SemaphoreType.DMA,
                       pltpu.SemaphoreType.DMA,
                       pltpu.SemaphoreType.DMA,
                       pltpu.SemaphoreType.DMA,
                       pltpu.SemaphoreType.DMA,
                       pltpu.SemaphoreType.DMA],
    )(_gather_body)
    return f(ps, pr, senders, receivers)


def _edge_body(e_ref, g_ref, we_ref, b1_ref, w2_ref, b2_ref, o_ref):
    x = jnp.dot(e_ref[...], we_ref[...], preferred_element_type=jnp.float32)
    h = jnp.maximum(x + g_ref[...] + b1_ref[...], 0.0)
    o_ref[...] = (jnp.dot(h, w2_ref[...], preferred_element_type=jnp.float32)
                  + b2_ref[...])


def _edge_mlp(edges, gsum, we, be1, we2, be2):
    eb = 1280
    blk = lambda i: (i, 0)
    wspec = pl.BlockSpec((_D, _D), lambda i: (0, 0))
    bspec = pl.BlockSpec((1, _D), lambda i: (0, 0))
    return pl.pallas_call(
        _edge_body,
        grid=(_E // eb,),
        in_specs=[pl.BlockSpec((eb, _D), blk),
                  pl.BlockSpec((eb, _D), blk),
                  wspec, bspec, wspec, bspec],
        out_specs=pl.BlockSpec((eb, _D), blk),
        out_shape=jax.ShapeDtypeStruct((_E, _D), jnp.float32),
    )(edges, gsum, we, be1.reshape(1, _D), we2, be2.reshape(1, _D))


def _scatter_body(ne_hbm, rcv_hbm, zeros_hbm, sums_hbm,
                  idx0, idx1, idxt, rows, acc_s,
                  sem_l0, sem_l1, sem_a0, sem_a1):
    cid = lax.axis_index("c")
    sid = lax.axis_index("s")
    wid = sid * 2 + cid
    srow = sid * _STRIPE
    idx_v = (idx0, idx1)
    sem_l = (sem_l0, sem_l1)
    sem_a = (sem_a0, sem_a1)
    pltpu.sync_copy(zeros_hbm.at[pl.ds(srow, _STRIPE)],
                    acc_s.at[pl.ds(srow, _STRIPE)])
    plsc.subcore_barrier()

    base0 = wid * _EPT

    def issue_loads(g):
        b = g & 1
        base = base0 + g * _C2
        return [pltpu.async_copy(ne_hbm.at[pl.ds(base, _C2)], rows.at[b],
                                 sem_l[b]),
                pltpu.async_copy(rcv_hbm.at[pl.ds(base, _C2)], idx_v[b],
                                 sem_l[b])]

    h_l = [None, None]
    h_a = [None, None]
    h_l[0] = issue_loads(0)
    for g in range(_NC2):
        b = g & 1
        for h in h_l[b]:
            h.wait()
        if g + 1 < _NC2:
            if h_a[1 - b] is not None:
                h_a[1 - b].wait()
            h_l[1 - b] = issue_loads(g + 1)
        h_a[b] = pltpu.async_copy(rows.at[b], acc_s.at[idx_v[b]], sem_a[b],
                                  add=True)
    for ha in h_a:
        if ha is not None:
            ha.wait()

    tbase = base0 + _NC2 * _C2
    pltpu.sync_copy(rcv_hbm.at[pl.ds(tbase, _TAIL)], idxt)
    pltpu.sync_copy(ne_hbm.at[pl.ds(tbase, _TAIL)], rows.at[0, pl.ds(0, _TAIL)])
    pltpu.sync_copy(rows.at[0, pl.ds(0, _TAIL)], acc_s.at[idxt], add=True)

    plsc.subcore_barrier()
    pltpu.sync_copy(acc_s.at[pl.ds(srow, _STRIPE)],
                    sums_hbm.at[cid, pl.ds(srow, _STRIPE)])


def _scatter(new_edges, receivers, zeros):
    mesh = plsc.VectorSubcoreMesh(core_axis_name="c", subcore_axis_name="s")
    f = functools.partial(
        pl.kernel,
        out_type=jax.ShapeDtypeStruct((2, _NP, _D), jnp.float32),
        mesh=mesh,
        scratch_types=[pltpu.VMEM((_C2,), jnp.int32),
                       pltpu.VMEM((_C2,), jnp.int32),
                       pltpu.VMEM((_TAIL,), jnp.int32),
                       pltpu.VMEM((2, _C2, _D), jnp.float32),
                       pltpu.VMEM_SHARED((_NP, _D), jnp.float32),
                       pltpu.SemaphoreType.DMA,
                       pltpu.SemaphoreType.DMA,
                       pltpu.SemaphoreType.DMA,
                       pltpu.SemaphoreType.DMA],
    )(_scatter_body)
    return f(new_edges, receivers, zeros)


def _counts_body(rcv_hbm, zeros_hbm, ones_hbm, cnts_hbm,
                 idx0, idx1, idxt, ones_v, acc_c,
                 sem_i0, sem_i1, sem_a0, sem_a1):
    cid = lax.axis_index("c")
    sid = lax.axis_index("s")
    wid = sid * 2 + cid
    srow = sid * _STRIPE
    idx_v = (idx0, idx1)
    sem_i = (sem_i0, sem_i1)
    sem_a = (sem_a0, sem_a1)
    pltpu.sync_copy(zeros_hbm.at[pl.ds(srow, _STRIPE)],
                    acc_c.at[pl.ds(srow, _STRIPE)])
    pltpu.sync_copy(ones_hbm, ones_v)
    plsc.subcore_barrier()

    base0 = wid * _EPT

    def issue_idx(g):
        b = g & 1
        base = base0 + g * _C2
        return [pltpu.async_copy(rcv_hbm.at[pl.ds(base, _C2)], idx_v[b],
                                 sem_i[b])]

    h_i = [None, None]
    h_a = [None, None]
    h_i[0] = issue_idx(0)
    for g in range(_NC2):
        b = g & 1
        for h in h_i[b]:
            h.wait()
        if g + 1 < _NC2:
            if h_a[1 - b] is not None:
                h_a[1 - b].wait()
            h_i[1 - b] = issue_idx(g + 1)
        h_a[b] = pltpu.async_copy(ones_v, acc_c.at[idx_v[b]], sem_a[b],
                                  add=True)
    for ha in h_a:
        if ha is not None:
            ha.wait()

    tbase = base0 + _NC2 * _C2
    pltpu.sync_copy(rcv_hbm.at[pl.ds(tbase, _TAIL)], idxt)
    pltpu.sync_copy(ones_v.at[pl.ds(0, _TAIL)], acc_c.at[idxt], add=True)

    plsc.subcore_barrier()
    pltpu.sync_copy(acc_c.at[pl.ds(srow, _STRIPE)],
                    cnts_hbm.at[cid, pl.ds(srow, _STRIPE)])


def _counts(receivers, zeros, ones):
    mesh = plsc.VectorSubcoreMesh(core_axis_name="c", subcore_axis_name="s")
    f = functools.partial(
        pl.kernel,
        out_type=jax.ShapeDtypeStruct((2, _NP, _D), jnp.float32),
        mesh=mesh,
        scratch_types=[pltpu.VMEM((_C2,), jnp.int32),
                       pltpu.VMEM((_C2,), jnp.int32),
                       pltpu.VMEM((_TAIL,), jnp.int32),
                       pltpu.VMEM((_C2, _D), jnp.float32),
                       pltpu.VMEM_SHARED((_NP, _D), jnp.float32),
                       pltpu.SemaphoreType.DMA,
                       pltpu.SemaphoreType.DMA,
                       pltpu.SemaphoreType.DMA,
                       pltpu.SemaphoreType.DMA],
    )(_counts_body)
    return f(receivers, zeros, ones)


def _node_body(s0_ref, s1_ref, c0_ref, c1_ref, pn_ref, w_ref, b1_ref,
               w2_ref, b2_ref, o_ref):
    sums = s0_ref[...] + s1_ref[...]
    cnt = jnp.max(c0_ref[...] + c1_ref[...], axis=1, keepdims=True)
    agg = sums / jnp.maximum(cnt, 1.0)
    hn = jnp.maximum(
        pn_ref[...]
        + jnp.dot(agg, w_ref[...], preferred_element_type=jnp.float32)
        + b1_ref[...], 0.0)
    o_ref[...] = (jnp.dot(hn, w2_ref[...], preferred_element_type=jnp.float32)
                  + b2_ref[...])


def _node_mlp(s0, s1, c0, c1, pn, wn1b, bn1, wn2, bn2):
    nb = 1000
    blk = lambda i: (i, 0)
    wspec = pl.BlockSpec((_D, _D), lambda i: (0, 0))
    bspec = pl.BlockSpec((1, _D), lambda i: (0, 0))
    return pl.pallas_call(
        _node_body,
        grid=(_N // nb,),
        in_specs=[pl.BlockSpec((nb, _D), blk),
                  pl.BlockSpec((nb, _D), blk),
                  pl.BlockSpec((nb, _D), blk),
                  pl.BlockSpec((nb, _D), blk),
                  pl.BlockSpec((nb, _D), blk),
                  wspec, bspec, wspec, bspec],
        out_specs=pl.BlockSpec((nb, _D), blk),
        out_shape=jax.ShapeDtypeStruct((_N, _D), jnp.float32),
    )(s0, s1, c0, c1, pn, wn1b, bn1.reshape(1, _D), wn2, bn2.reshape(1, _D))


def kernel(nodes, edges, senders, receivers,
           We1, be1, We2, be2, Wn1, bn1, Wn2, bn2):
    we = We1[0:_D]
    ws = We1[_D:2 * _D]
    wr = We1[2 * _D:3 * _D]
    wn1a = Wn1[0:_D]
    wn1b = Wn1[_D:2 * _D]

    ps, pr, pn = _proj(nodes, jnp.concatenate([ws, wr, wn1a], axis=1))

    gsum = _gather(ps, pr, senders, receivers)
    new_edges = _edge_mlp(edges, gsum, we, be1, We2, be2)

    zeros = jnp.zeros((_NP, _D), jnp.float32)
    ones = jnp.ones((_C2, _D), jnp.float32)
    cnts = _counts(receivers, zeros, ones)
    sums = _scatter(new_edges, receivers, zeros)

    new_nodes = _node_mlp(sums[0, :_N], sums[1, :_N], cnts[0, :_N],
                          cnts[1, :_N], pn, wn1b, bn1, Wn2, bn2)
    return (new_nodes, new_edges)


# speedup vs baseline: 1.2348x; 1.1150x over previous
import functools

import jax
import jax.numpy as jnp
from jax import lax
from jax.experimental import pallas as pl
from jax.experimental.pallas import tpu as pltpu
from jax.experimental.pallas import tpu_sc as plsc

_N = 10000
_E = 320000
_D = 128
_H = _D // 2

_NW = 32
_EPT = _E // _NW
_S = 400
_SUB = 80
_K = _S // _SUB
_NO = _EPT // _S
_NP = 10240
_STRIPE = _NP // 16
_C2 = 128
_NC2 = _EPT // _C2
_TAIL = _EPT - _NC2 * _C2


def _proj_body(n_ref, w_ref, ps_ref, pr_ref, pn_ref):
    o = jnp.dot(n_ref[...], w_ref[...], preferred_element_type=jnp.float32)
    ps_ref[...] = o[:, 0:_D]
    pr_ref[...] = o[:, _D:2 * _D]
    pn_ref[...] = o[:, 2 * _D:3 * _D]


def _proj(nodes, wcat):
    nb = 2000
    blk = lambda i: (i, 0)
    return pl.pallas_call(
        _proj_body,
        grid=(_N // nb,),
        in_specs=[pl.BlockSpec((nb, _D), blk),
                  pl.BlockSpec((_D, 3 * _D), lambda i: (0, 0))],
        out_specs=[pl.BlockSpec((nb, _D), blk),
                   pl.BlockSpec((nb, _D), blk),
                   pl.BlockSpec((nb, _D), blk)],
        out_shape=[jax.ShapeDtypeStruct((_N, _D), jnp.float32),
                   jax.ShapeDtypeStruct((_N, _D), jnp.float32),
                   jax.ShapeDtypeStruct((_N, _D), jnp.float32)],
    )(nodes, wcat)


_GS = 120
_GNO = _EPT // _GS
_GT = _EPT - _GNO * _GS


def _gather_body(ps_hbm, pr_hbm, snd_hbm, rcv_hbm, o_hbm,
                 si0, si1, ri0, ri1, pbuf, rbuf,
                 sem_i0, sem_i1, sem_g0, sem_g1, sem_w0, sem_w1):
    wid = lax.axis_index("s") * 2 + lax.axis_index("c")
    base0 = wid * _EPT
    si = (si0, si1)
    ri = (ri0, ri1)
    sem_i = (sem_i0, sem_i1)
    sem_g = (sem_g0, sem_g1)
    sem_w = (sem_w0, sem_w1)

    def issue_idx(g):
        b = g & 1
        base = base0 + g * _GS
        return [pltpu.async_copy(snd_hbm.at[pl.ds(base, _GS)], si[b],
                                 sem_i[b]),
                pltpu.async_copy(rcv_hbm.at[pl.ds(base, _GS)], ri[b],
                                 sem_i[b])]

    def issue_gathers(g):
        b = g & 1
        return [pltpu.async_copy(ps_hbm.at[si[b]], pbuf.at[b], sem_g[b]),
                pltpu.async_copy(pr_hbm.at[ri[b]], rbuf.at[b], sem_g[b])]

    def add_rows(b):
        def body(i, carry):
            for j in range(_D // 16):
                sl = pl.ds(j * 16, 16)
                pbuf[b, i, sl] = pbuf[b, i, sl] + rbuf[b, i, sl]
            return carry
        lax.fori_loop(0, _GS, body, 0)

    def issue_writeback(g):
        b = g & 1
        base = base0 + g * _GS
        return [pltpu.async_copy(pbuf.at[b], o_hbm.at[pl.ds(base, _GS)],
                                 sem_w[b])]

    h_i = [None, None]
    h_g = [None, None]
    h_w = [None, None]
    h_i[0] = issue_idx(0)
    for g in range(_GNO):
        b = g & 1
        if h_w[b] is not None:
            for h in h_w[b]:
                h.wait()
        for h in h_i[b]:
            h.wait()
        h_g[b] = issue_gathers(g)
        if g >= 1:
            for h in h_g[1 - b]:
                h.wait()
            add_rows(1 - b)
            h_w[1 - b] = issue_writeback(g - 1)
            if g + 1 < _GNO:
                h_i[1 - b] = issue_idx(g + 1)
        else:
            h_i[1] = issue_idx(1)
    bl = (_GNO - 1) & 1
    for h in h_g[bl]:
        h.wait()
    add_rows(bl)
    h_w[bl] = issue_writeback(_GNO - 1)
    for hw in h_w:
        if hw is not None:
            for h in hw:
                h.wait()

    tbase = base0 + _GNO * _GS
    tsl = pl.ds(0, _GT)
    pltpu.sync_copy(snd_hbm.at[pl.ds(tbase, _GT)], si0.at[tsl])
    pltpu.sync_copy(rcv_hbm.at[pl.ds(tbase, _GT)], ri0.at[tsl])
    pltpu.async_copy(ps_hbm.at[si0.at[tsl]], pbuf.at[0, tsl], sem_g0).wait()
    pltpu.async_copy(pr_hbm.at[ri0.at[tsl]], rbuf.at[0, tsl], sem_g0).wait()

    def tbody(i, carry):
        for j in range(_D // 16):
            sl = pl.ds(j * 16, 16)
            pbuf[0, i, sl] = pbuf[0, i, sl] + rbuf[0, i, sl]
        return carry
    lax.fori_loop(0, _GT, tbody, 0)
    pltpu.sync_copy(pbuf.at[0, tsl], o_hbm.at[pl.ds(tbase, _GT)])


def _gather(ps, pr, senders, receivers):
    mesh = plsc.VectorSubcoreMesh(core_axis_name="c", subcore_axis_name="s")
    f = functools.partial(
        pl.kernel,
        out_type=jax.ShapeDtypeStruct((_E, _D), jnp.float32),
        mesh=mesh,
        scratch_types=[pltpu.VMEM((_GS,), jnp.int32),
                       pltpu.VMEM((_GS,), jnp.int32),
                       pltpu.VMEM((_GS,), jnp.int32),
                       pltpu.VMEM((_GS,), jnp.int32),
                       pltpu.VMEM((2, _GS, _D), jnp.float32),
                       pltpu.VMEM((2, _GS, _D), jnp.float32),
                       pltpu.SemaphoreType.DMA,
                       pltpu.SemaphoreType.DMA,
                       pltpu.SemaphoreType.DMA,
                       pltpu.SemaphoreType.DMA,
                       pltpu.SemaphoreType.DMA,
                       pltpu.SemaphoreType.DMA],
    )(_gather_body)
    return f(ps, pr, senders, receivers)


def _edge_body(e_ref, g_ref, we_ref, b1_ref, w2_ref, b2_ref, o_ref):
    x = jnp.dot(e_ref[...], we_ref[...], preferred_element_type=jnp.float32)
    h = jnp.maximum(x + g_ref[...] + b1_ref[...], 0.0)
    o_ref[...] = (jnp.dot(h, w2_ref[...], preferred_element_type=jnp.float32)
                  + b2_ref[...])


def _edge_mlp(edges, gsum, we, be1, we2, be2):
    eb = 2000
    blk = lambda i: (i, 0)
    wspec = pl.BlockSpec((_D, _D), lambda i: (0, 0))
    bspec = pl.BlockSpec((1, _D), lambda i: (0, 0))
    return pl.pallas_call(
        _edge_body,
        grid=(_E // eb,),
        in_specs=[pl.BlockSpec((eb, _D), blk),
                  pl.BlockSpec((eb, _D), blk),
                  wspec, bspec, wspec, bspec],
        out_specs=pl.BlockSpec((eb, _D), blk),
        out_shape=jax.ShapeDtypeStruct((_E, _D), jnp.float32),
    )(edges, gsum, we, be1.reshape(1, _D), we2, be2.reshape(1, _D))


def _scatter_body(ne_hbm, rcv_hbm, zeros_hbm, sums_hbm,
                  idx0, idx1, idxt, rows, acc_s,
                  sem_l0, sem_l1, sem_a0, sem_a1):
    cid = lax.axis_index("c")
    sid = lax.axis_index("s")
    wid = sid * 2 + cid
    srow = sid * _STRIPE
    idx_v = (idx0, idx1)
    sem_l = (sem_l0, sem_l1)
    sem_a = (sem_a0, sem_a1)
    pltpu.sync_copy(zeros_hbm.at[pl.ds(srow, _STRIPE)],
                    acc_s.at[pl.ds(srow, _STRIPE)])
    plsc.subcore_barrier()

    base0 = wid * _EPT

    def issue_loads(g):
        b = g & 1
        base = base0 + g * _C2
        return [pltpu.async_copy(ne_hbm.at[pl.ds(base, _C2)], rows.at[b],
                                 sem_l[b]),
                pltpu.async_copy(rcv_hbm.at[pl.ds(base, _C2)], idx_v[b],
                                 sem_l[b])]

    h_l = [None, None]
    h_a = [None, None]
    h_l[0] = issue_loads(0)
    for g in range(_NC2):
        b = g & 1
        for h in h_l[b]:
            h.wait()
        if g + 1 < _NC2:
            if h_a[1 - b] is not None:
                h_a[1 - b].wait()
            h_l[1 - b] = issue_loads(g + 1)
        h_a[b] = pltpu.async_copy(rows.at[b], acc_s.at[idx_v[b]], sem_a[b],
                                  add=True)
    for ha in h_a:
        if ha is not None:
            ha.wait()

    tbase = base0 + _NC2 * _C2
    pltpu.sync_copy(rcv_hbm.at[pl.ds(tbase, _TAIL)], idxt)
    pltpu.sync_copy(ne_hbm.at[pl.ds(tbase, _TAIL)], rows.at[0, pl.ds(0, _TAIL)])
    pltpu.sync_copy(rows.at[0, pl.ds(0, _TAIL)], acc_s.at[idxt], add=True)

    plsc.subcore_barrier()
    pltpu.sync_copy(acc_s.at[pl.ds(srow, _STRIPE)],
                    sums_hbm.at[cid, pl.ds(srow, _STRIPE)])


def _scatter(new_edges, receivers, zeros):
    mesh = plsc.VectorSubcoreMesh(core_axis_name="c", subcore_axis_name="s")
    f = functools.partial(
        pl.kernel,
        out_type=jax.ShapeDtypeStruct((2, _NP, _D), jnp.float32),
        mesh=mesh,
        scratch_types=[pltpu.VMEM((_C2,), jnp.int32),
                       pltpu.VMEM((_C2,), jnp.int32),
                       pltpu.VMEM((_TAIL,), jnp.int32),
                       pltpu.VMEM((2, _C2, _D), jnp.float32),
                       pltpu.VMEM_SHARED((_NP, _D), jnp.float32),
                       pltpu.SemaphoreType.DMA,
                       pltpu.SemaphoreType.DMA,
                       pltpu.SemaphoreType.DMA,
                       pltpu.SemaphoreType.DMA],
    )(_scatter_body)
    return f(new_edges, receivers, zeros)


def _counts_body(rcv_hbm, zeros_hbm, ones_hbm, cnts_hbm,
                 idx0, idx1, idxt, ones_v, acc_c,
                 sem_i0, sem_i1, sem_a0, sem_a1):
    cid = lax.axis_index("c")
    sid = lax.axis_index("s")
    wid = sid * 2 + cid
    srow = sid * _STRIPE
    idx_v = (idx0, idx1)
    sem_i = (sem_i0, sem_i1)
    sem_a = (sem_a0, sem_a1)
    pltpu.sync_copy(zeros_hbm.at[pl.ds(srow, _STRIPE)],
                    acc_c.at[pl.ds(srow, _STRIPE)])
    pltpu.sync_copy(ones_hbm, ones_v)
    plsc.subcore_barrier()

    base0 = wid * _EPT

    def issue_idx(g):
        b = g & 1
        base = base0 + g * _C2
        return [pltpu.async_copy(rcv_hbm.at[pl.ds(base, _C2)], idx_v[b],
                                 sem_i[b])]

    h_i = [None, None]
    h_a = [None, None]
    h_i[0] = issue_idx(0)
    for g in range(_NC2):
        b = g & 1
        for h in h_i[b]:
            h.wait()
        if g + 1 < _NC2:
            if h_a[1 - b] is not None:
                h_a[1 - b].wait()
            h_i[1 - b] = issue_idx(g + 1)
        h_a[b] = pltpu.async_copy(ones_v, acc_c.at[idx_v[b]], sem_a[b],
                                  add=True)
    for ha in h_a:
        if ha is not None:
            ha.wait()

    tbase = base0 + _NC2 * _C2
    pltpu.sync_copy(rcv_hbm.at[pl.ds(tbase, _TAIL)], idxt)
    pltpu.sync_copy(ones_v.at[pl.ds(0, _TAIL)], acc_c.at[idxt], add=True)

    plsc.subcore_barrier()
    pltpu.sync_copy(acc_c.at[pl.ds(srow, _STRIPE)],
                    cnts_hbm.at[cid, pl.ds(srow, _STRIPE)])


def _counts(receivers, zeros, ones):
    mesh = plsc.VectorSubcoreMesh(core_axis_name="c", subcore_axis_name="s")
    f = functools.partial(
        pl.kernel,
        out_type=jax.ShapeDtypeStruct((2, _NP, _D), jnp.float32),
        mesh=mesh,
        scratch_types=[pltpu.VMEM((_C2,), jnp.int32),
                       pltpu.VMEM((_C2,), jnp.int32),
                       pltpu.VMEM((_TAIL,), jnp.int32),
                       pltpu.VMEM((_C2, _D), jnp.float32),
                       pltpu.VMEM_SHARED((_NP, _D), jnp.float32),
                       pltpu.SemaphoreType.DMA,
                       pltpu.SemaphoreType.DMA,
                       pltpu.SemaphoreType.DMA,
                       pltpu.SemaphoreType.DMA],
    )(_counts_body)
    return f(receivers, zeros, ones)


def _node_body(s0_ref, s1_ref, c0_ref, c1_ref, pn_ref, w_ref, b1_ref,
               w2_ref, b2_ref, o_ref):
    sums = s0_ref[0] + s1_ref[0]
    cnt = jnp.max(c0_ref[0] + c1_ref[0], axis=1, keepdims=True)
    agg = sums / jnp.maximum(cnt, 1.0)
    hn = jnp.maximum(
        pn_ref[...]
        + jnp.dot(agg, w_ref[...], preferred_element_type=jnp.float32)
        + b1_ref[...], 0.0)
    o_ref[...] = (jnp.dot(hn, w2_ref[...], preferred_element_type=jnp.float32)
                  + b2_ref[...])


def _node_mlp(sums, cnts, pn, wn1b, bn1, wn2, bn2):
    nb = 1000
    blk = lambda i: (i, 0)
    j0 = pl.BlockSpec((1, nb, _D), lambda i: (0, i, 0))
    j1 = pl.BlockSpec((1, nb, _D), lambda i: (1, i, 0))
    wspec = pl.BlockSpec((_D, _D), lambda i: (0, 0))
    bspec = pl.BlockSpec((1, _D), lambda i: (0, 0))
    return pl.pallas_call(
        _node_body,
        grid=(_N // nb,),
        in_specs=[j0, j1, j0, j1,
                  pl.BlockSpec((nb, _D), blk),
                  wspec, bspec, wspec, bspec],
        out_specs=pl.BlockSpec((nb, _D), blk),
        out_shape=jax.ShapeDtypeStruct((_N, _D), jnp.float32),
    )(sums, sums, cnts, cnts, pn, wn1b, bn1.reshape(1, _D), wn2,
      bn2.reshape(1, _D))


def kernel(nodes, edges, senders, receivers,
           We1, be1, We2, be2, Wn1, bn1, Wn2, bn2):
    we = We1[0:_D]
    ws = We1[_D:2 * _D]
    wr = We1[2 * _D:3 * _D]
    wn1a = Wn1[0:_D]
    wn1b = Wn1[_D:2 * _D]

    ps, pr, pn = _proj(nodes, jnp.concatenate([ws, wr, wn1a], axis=1))

    gsum = _gather(ps, pr, senders, receivers)
    new_edges = _edge_mlp(edges, gsum, we, be1, We2, be2)

    zeros = jnp.zeros((_NP, _D), jnp.float32)
    ones = jnp.ones((_C2, _D), jnp.float32)
    cnts = _counts(receivers, zeros, ones)
    sums = _scatter(new_edges, receivers, zeros)

    new_nodes = _node_mlp(sums, cnts, pn, wn1b, bn1, Wn2, bn2)
    return (new_nodes, new_edges)


# speedup vs baseline: 1.2803x; 1.0369x over previous
import functools

import jax
import jax.numpy as jnp
from jax import lax
from jax.experimental import pallas as pl
from jax.experimental.pallas import tpu as pltpu
from jax.experimental.pallas import tpu_sc as plsc

_N = 10000
_E = 320000
_D = 128
_H = _D // 2

_NW = 32
_EPT = _E // _NW
_S = 400
_SUB = 80
_K = _S // _SUB
_NO = _EPT // _S
_NP = 10240
_STRIPE = _NP // 16
_C2 = 128
_NC2 = _EPT // _C2
_TAIL = _EPT - _NC2 * _C2


def _proj_body(n_ref, w_ref, ps_ref, pr_ref, pn_ref):
    o = jnp.dot(n_ref[...], w_ref[...], preferred_element_type=jnp.float32)
    ps_ref[...] = o[:, 0:_D]
    pr_ref[...] = o[:, _D:2 * _D]
    pn_ref[...] = o[:, 2 * _D:3 * _D]


def _proj(nodes, wcat):
    nb = 2000
    blk = lambda i: (i, 0)
    return pl.pallas_call(
        _proj_body,
        grid=(_N // nb,),
        in_specs=[pl.BlockSpec((nb, _D), blk),
                  pl.BlockSpec((_D, 3 * _D), lambda i: (0, 0))],
        out_specs=[pl.BlockSpec((nb, _D), blk),
                   pl.BlockSpec((nb, _D), blk),
                   pl.BlockSpec((nb, _D), blk)],
        out_shape=[jax.ShapeDtypeStruct((_N, _D), jnp.float32),
                   jax.ShapeDtypeStruct((_N, _D), jnp.float32),
                   jax.ShapeDtypeStruct((_N, _D), jnp.float32)],
    )(nodes, wcat)


_GS = 120
_GNO = _EPT // _GS
_GT = _EPT - _GNO * _GS


def _gather_body(ps_hbm, pr_hbm, snd_hbm, rcv_hbm, o_hbm,
                 si0, si1, ri0, ri1, pbuf, rbuf,
                 sem_i0, sem_i1, sem_g0, sem_g1, sem_w0, sem_w1):
    wid = lax.axis_index("s") * 2 + lax.axis_index("c")
    base0 = wid * _EPT
    si = (si0, si1)
    ri = (ri0, ri1)
    sem_i = (sem_i0, sem_i1)
    sem_g = (sem_g0, sem_g1)
    sem_w = (sem_w0, sem_w1)

    def issue_idx(g):
        b = g & 1
        base = base0 + g * _GS
        return [pltpu.async_copy(snd_hbm.at[pl.ds(base, _GS)], si[b],
                                 sem_i[b]),
                pltpu.async_copy(rcv_hbm.at[pl.ds(base, _GS)], ri[b],
                                 sem_i[b])]

    def issue_gathers(g):
        b = g & 1
        return [pltpu.async_copy(ps_hbm.at[si[b]], pbuf.at[b], sem_g[b]),
                pltpu.async_copy(pr_hbm.at[ri[b]], rbuf.at[b], sem_g[b])]

    def add_rows(b):
        def body(i, carry):
            for j in range(_D // 16):
                sl = pl.ds(j * 16, 16)
                pbuf[b, i, sl] = pbuf[b, i, sl] + rbuf[b, i, sl]
            return carry
        lax.fori_loop(0, _GS, body, 0)

    def issue_writeback(g):
        b = g & 1
        base = base0 + g * _GS
        return [pltpu.async_copy(pbuf.at[b], o_hbm.at[pl.ds(base, _GS)],
                                 sem_w[b])]

    h_i = [None, None]
    h_g = [None, None]
    h_w = [None, None]
    h_i[0] = issue_idx(0)
    for g in range(_GNO):
        b = g & 1
        if h_w[b] is not None:
            for h in h_w[b]:
                h.wait()
        for h in h_i[b]:
            h.wait()
        h_g[b] = issue_gathers(g)
        if g >= 1:
            for h in h_g[1 - b]:
                h.wait()
            add_rows(1 - b)
            h_w[1 - b] = issue_writeback(g - 1)
            if g + 1 < _GNO:
                h_i[1 - b] = issue_idx(g + 1)
        else:
            h_i[1] = issue_idx(1)
    bl = (_GNO - 1) & 1
    for h in h_g[bl]:
        h.wait()
    add_rows(bl)
    h_w[bl] = issue_writeback(_GNO - 1)
    for hw in h_w:
        if hw is not None:
            for h in hw:
                h.wait()

    tbase = base0 + _GNO * _GS
    tsl = pl.ds(0, _GT)
    pltpu.sync_copy(snd_hbm.at[pl.ds(tbase, _GT)], si0.at[tsl])
    pltpu.sync_copy(rcv_hbm.at[pl.ds(tbase, _GT)], ri0.at[tsl])
    pltpu.async_copy(ps_hbm.at[si0.at[tsl]], pbuf.at[0, tsl], sem_g0).wait()
    pltpu.async_copy(pr_hbm.at[ri0.at[tsl]], rbuf.at[0, tsl], sem_g0).wait()

    def tbody(i, carry):
        for j in range(_D // 16):
            sl = pl.ds(j * 16, 16)
            pbuf[0, i, sl] = pbuf[0, i, sl] + rbuf[0, i, sl]
        return carry
    lax.fori_loop(0, _GT, tbody, 0)
    pltpu.sync_copy(pbuf.at[0, tsl], o_hbm.at[pl.ds(tbase, _GT)])


def _gather(ps, pr, senders, receivers):
    mesh = plsc.VectorSubcoreMesh(core_axis_name="c", subcore_axis_name="s")
    f = functools.partial(
        pl.kernel,
        out_type=jax.ShapeDtypeStruct((_E, _D), jnp.float32),
        mesh=mesh,
        scratch_types=[pltpu.VMEM((_GS,), jnp.int32),
                       pltpu.VMEM((_GS,), jnp.int32),
                       pltpu.VMEM((_GS,), jnp.int32),
                       pltpu.VMEM((_GS,), jnp.int32),
                       pltpu.VMEM((2, _GS, _D), jnp.float32),
                       pltpu.VMEM((2, _GS, _D), jnp.float32),
                       pltpu.SemaphoreType.DMA,
                       pltpu.SemaphoreType.DMA,
                       pltpu.SemaphoreType.DMA,
                       pltpu.SemaphoreType.DMA,
                       pltpu.SemaphoreType.DMA,
                       pltpu.SemaphoreType.DMA],
    )(_gather_body)
    return f(ps, pr, senders, receivers)


def _edge_body(e_ref, g_ref, we_ref, b1_ref, w2_ref, b2_ref, o_ref):
    x = jnp.dot(e_ref[...], we_ref[...], preferred_element_type=jnp.float32)
    h = jnp.maximum(x + g_ref[...] + b1_ref[...], 0.0)
    o_ref[...] = (jnp.dot(h, w2_ref[...], preferred_element_type=jnp.float32)
                  + b2_ref[...])


def _edge_mlp(edges, gsum, we, be1, we2, be2):
    eb = 2560
    blk = lambda i: (i, 0)
    wspec = pl.BlockSpec((_D, _D), lambda i: (0, 0))
    bspec = pl.BlockSpec((1, _D), lambda i: (0, 0))
    return pl.pallas_call(
        _edge_body,
        grid=(_E // eb,),
        in_specs=[pl.BlockSpec((eb, _D), blk),
                  pl.BlockSpec((eb, _D), blk),
                  wspec, bspec, wspec, bspec],
        out_specs=pl.BlockSpec((eb, _D), blk),
        out_shape=jax.ShapeDtypeStruct((_E, _D), jnp.float32),
    )(edges, gsum, we, be1.reshape(1, _D), we2, be2.reshape(1, _D))


def _scatter_body(ne_hbm, rcv_hbm, zeros_hbm, sums_hbm,
                  idx0, idx1, idxt, rows, acc_s,
                  sem_l0, sem_l1, sem_a0, sem_a1):
    cid = lax.axis_index("c")
    sid = lax.axis_index("s")
    wid = sid * 2 + cid
    srow = sid * _STRIPE
    idx_v = (idx0, idx1)
    sem_l = (sem_l0, sem_l1)
    sem_a = (sem_a0, sem_a1)
    pltpu.sync_copy(zeros_hbm.at[pl.ds(srow, _STRIPE)],
                    acc_s.at[pl.ds(srow, _STRIPE)])
    plsc.subcore_barrier()

    base0 = wid * _EPT

    def issue_loads(g):
        b = g & 1
        base = base0 + g * _C2
        return [pltpu.async_copy(ne_hbm.at[pl.ds(base, _C2)], rows.at[b],
                                 sem_l[b]),
                pltpu.async_copy(rcv_hbm.at[pl.ds(base, _C2)], idx_v[b],
                                 sem_l[b])]

    h_l = [None, None]
    h_a = [None, None]
    h_l[0] = issue_loads(0)
    for g in range(_NC2):
        b = g & 1
        for h in h_l[b]:
            h.wait()
        if g + 1 < _NC2:
            if h_a[1 - b] is not None:
                h_a[1 - b].wait()
            h_l[1 - b] = issue_loads(g + 1)
        h_a[b] = pltpu.async_copy(rows.at[b], acc_s.at[idx_v[b]], sem_a[b],
                                  add=True)
    for ha in h_a:
        if ha is not None:
            ha.wait()

    tbase = base0 + _NC2 * _C2
    pltpu.sync_copy(rcv_hbm.at[pl.ds(tbase, _TAIL)], idxt)
    pltpu.sync_copy(ne_hbm.at[pl.ds(tbase, _TAIL)], rows.at[0, pl.ds(0, _TAIL)])
    pltpu.sync_copy(rows.at[0, pl.ds(0, _TAIL)], acc_s.at[idxt], add=True)

    plsc.subcore_barrier()
    pltpu.sync_copy(acc_s.at[pl.ds(srow, _STRIPE)],
                    sums_hbm.at[cid, pl.ds(srow, _STRIPE)])


def _scatter(new_edges, receivers, zeros):
    mesh = plsc.VectorSubcoreMesh(core_axis_name="c", subcore_axis_name="s")
    f = functools.partial(
        pl.kernel,
        out_type=jax.ShapeDtypeStruct((2, _NP, _D), jnp.float32),
        mesh=mesh,
        scratch_types=[pltpu.VMEM((_C2,), jnp.int32),
                       pltpu.VMEM((_C2,), jnp.int32),
                       pltpu.VMEM((_TAIL,), jnp.int32),
                       pltpu.VMEM((2, _C2, _D), jnp.float32),
                       pltpu.VMEM_SHARED((_NP, _D), jnp.float32),
                       pltpu.SemaphoreType.DMA,
                       pltpu.SemaphoreType.DMA,
                       pltpu.SemaphoreType.DMA,
                       pltpu.SemaphoreType.DMA],
    )(_scatter_body)
    return f(new_edges, receivers, zeros)


def _counts_body(rcv_hbm, zeros_hbm, ones_hbm, cnts_hbm,
                 idx0, idx1, idxt, ones_v, acc_c,
                 sem_i0, sem_i1, sem_a0, sem_a1):
    cid = lax.axis_index("c")
    sid = lax.axis_index("s")
    wid = sid * 2 + cid
    srow = sid * _STRIPE
    idx_v = (idx0, idx1)
    sem_i = (sem_i0, sem_i1)
    sem_a = (sem_a0, sem_a1)
    pltpu.sync_copy(zeros_hbm.at[pl.ds(srow, _STRIPE)],
                    acc_c.at[pl.ds(srow, _STRIPE)])
    pltpu.sync_copy(ones_hbm, ones_v)
    plsc.subcore_barrier()

    base0 = wid * _EPT

    def issue_idx(g):
        b = g & 1
        base = base0 + g * _C2
        return [pltpu.async_copy(rcv_hbm.at[pl.ds(base, _C2)], idx_v[b],
                                 sem_i[b])]

    h_i = [None, None]
    h_a = [None, None]
    h_i[0] = issue_idx(0)
    for g in range(_NC2):
        b = g & 1
        for h in h_i[b]:
            h.wait()
        if g + 1 < _NC2:
            if h_a[1 - b] is not None:
                h_a[1 - b].wait()
            h_i[1 - b] = issue_idx(g + 1)
        h_a[b] = pltpu.async_copy(ones_v, acc_c.at[idx_v[b]], sem_a[b],
                                  add=True)
    for ha in h_a:
        if ha is not None:
            ha.wait()

    tbase = base0 + _NC2 * _C2
    pltpu.sync_copy(rcv_hbm.at[pl.ds(tbase, _TAIL)], idxt)
    pltpu.sync_copy(ones_v.at[pl.ds(0, _TAIL)], acc_c.at[idxt], add=True)

    plsc.subcore_barrier()
    pltpu.sync_copy(acc_c.at[pl.ds(srow, _STRIPE)],
                    cnts_hbm.at[cid, pl.ds(srow, _STRIPE)])


def _counts(receivers, zeros, ones):
    mesh = plsc.VectorSubcoreMesh(core_axis_name="c", subcore_axis_name="s")
    f = functools.partial(
        pl.kernel,
        out_type=jax.ShapeDtypeStruct((2, _NP, _D), jnp.float32),
        mesh=mesh,
        scratch_types=[pltpu.VMEM((_C2,), jnp.int32),
                       pltpu.VMEM((_C2,), jnp.int32),
                       pltpu.VMEM((_TAIL,), jnp.int32),
                       pltpu.VMEM((_C2, _D), jnp.float32),
                       pltpu.VMEM_SHARED((_NP, _D), jnp.float32),
                       pltpu.SemaphoreType.DMA,
                       pltpu.SemaphoreType.DMA,
                       pltpu.SemaphoreType.DMA,
                       pltpu.SemaphoreType.DMA],
    )(_counts_body)
    return f(receivers, zeros, ones)


def _node_body(s0_ref, s1_ref, c0_ref, c1_ref, pn_ref, w_ref, b1_ref,
               w2_ref, b2_ref, o_ref):
    sums = s0_ref[0] + s1_ref[0]
    cnt = jnp.max(c0_ref[0] + c1_ref[0], axis=1, keepdims=True)
    agg = sums / jnp.maximum(cnt, 1.0)
    hn = jnp.maximum(
        pn_ref[...]
        + jnp.dot(agg, w_ref[...], preferred_element_type=jnp.float32)
        + b1_ref[...], 0.0)
    o_ref[...] = (jnp.dot(hn, w2_ref[...], preferred_element_type=jnp.float32)
                  + b2_ref[...])


def _node_mlp(sums, cnts, pn, wn1b, bn1, wn2, bn2):
    nb = 1000
    blk = lambda i: (i, 0)
    j0 = pl.BlockSpec((1, nb, _D), lambda i: (0, i, 0))
    j1 = pl.BlockSpec((1, nb, _D), lambda i: (1, i, 0))
    wspec = pl.BlockSpec((_D, _D), lambda i: (0, 0))
    bspec = pl.BlockSpec((1, _D), lambda i: (0, 0))
    return pl.pallas_call(
        _node_body,
        grid=(_N // nb,),
        in_specs=[j0, j1, j0, j1,
                  pl.BlockSpec((nb, _D), blk),
                  wspec, bspec, wspec, bspec],
        out_specs=pl.BlockSpec((nb, _D), blk),
        out_shape=jax.ShapeDtypeStruct((_N, _D), jnp.float32),
    )(sums, sums, cnts, cnts, pn, wn1b, bn1.reshape(1, _D), wn2,
      bn2.reshape(1, _D))


def kernel(nodes, edges, senders, receivers,
           We1, be1, We2, be2, Wn1, bn1, Wn2, bn2):
    we = We1[0:_D]
    ws = We1[_D:2 * _D]
    wr = We1[2 * _D:3 * _D]
    wn1a = Wn1[0:_D]
    wn1b = Wn1[_D:2 * _D]

    ps, pr, pn = _proj(nodes, jnp.concatenate([ws, wr, wn1a], axis=1))

    gsum = _gather(ps, pr, senders, receivers)
    new_edges = _edge_mlp(edges, gsum, we, be1, We2, be2)

    zeros = jnp.zeros((_NP, _D), jnp.float32)
    ones = jnp.ones((_C2, _D), jnp.float32)
    cnts = _counts(receivers, zeros, ones)
    sums = _scatter(new_edges, receivers, zeros)

    new_nodes = _node_mlp(sums, cnts, pn, wn1b, bn1, Wn2, bn2)
    return (new_nodes, new_edges)


# speedup vs baseline: 1.3543x; 1.0578x over previous
import functools

import jax
import jax.numpy as jnp
from jax import lax
from jax.experimental import pallas as pl
from jax.experimental.pallas import tpu as pltpu
from jax.experimental.pallas import tpu_sc as plsc

_N = 10000
_E = 320000
_D = 128
_H = _D // 2

_NW = 32
_EPT = _E // _NW
_S = 400
_SUB = 80
_K = _S // _SUB
_NO = _EPT // _S
_NP = 10240
_STRIPE = _NP // 16
_C2 = 128
_NC2 = _EPT // _C2
_TAIL = _EPT - _NC2 * _C2


def _proj_body(n_ref, w_ref, ps_ref, pr_ref, pn_ref):
    o = jnp.dot(n_ref[...], w_ref[...], preferred_element_type=jnp.float32)
    ps_ref[...] = o[:, 0:_D]
    pr_ref[...] = o[:, _D:2 * _D]
    pn_ref[...] = o[:, 2 * _D:3 * _D]


def _proj(nodes, wcat):
    nb = 2000
    blk = lambda i: (i, 0)
    return pl.pallas_call(
        _proj_body,
        grid=(_N // nb,),
        in_specs=[pl.BlockSpec((nb, _D), blk),
                  pl.BlockSpec((_D, 3 * _D), lambda i: (0, 0))],
        out_specs=[pl.BlockSpec((nb, _D), blk),
                   pl.BlockSpec((nb, _D), blk),
                   pl.BlockSpec((nb, _D), blk)],
        out_shape=[jax.ShapeDtypeStruct((_N, _D), jnp.float32),
                   jax.ShapeDtypeStruct((_N, _D), jnp.float32),
                   jax.ShapeDtypeStruct((_N, _D), jnp.float32)],
    )(nodes, wcat)


_GS = 120
_GNO = _EPT // _GS
_GT = _EPT - _GNO * _GS


def _gather_body(ps_hbm, pr_hbm, snd_hbm, rcv_hbm, o_hbm,
                 si0, si1, ri0, ri1, pbuf, rbuf,
                 sem_i0, sem_i1, sem_g0, sem_g1, sem_w0, sem_w1):
    wid = lax.axis_index("s") * 2 + lax.axis_index("c")
    base0 = wid * _EPT
    si = (si0, si1)
    ri = (ri0, ri1)
    sem_i = (sem_i0, sem_i1)
    sem_g = (sem_g0, sem_g1)
    sem_w = (sem_w0, sem_w1)

    def issue_idx(g):
        b = g & 1
        base = base0 + g * _GS
        return [pltpu.async_copy(snd_hbm.at[pl.ds(base, _GS)], si[b],
                                 sem_i[b]),
                pltpu.async_copy(rcv_hbm.at[pl.ds(base, _GS)], ri[b],
                                 sem_i[b])]

    def issue_gathers(g):
        b = g & 1
        return [pltpu.async_copy(ps_hbm.at[si[b]], pbuf.at[b], sem_g[b]),
                pltpu.async_copy(pr_hbm.at[ri[b]], rbuf.at[b], sem_g[b])]

    def add_rows(b):
        def body(i, carry):
            for j in range(_D // 16):
                sl = pl.ds(j * 16, 16)
                pbuf[b, i, sl] = pbuf[b, i, sl] + rbuf[b, i, sl]
            return carry
        lax.fori_loop(0, _GS, body, 0)

    def issue_writeback(g):
        b = g & 1
        base = base0 + g * _GS
        return [pltpu.async_copy(pbuf.at[b], o_hbm.at[pl.ds(base, _GS)],
                                 sem_w[b])]

    h_i = [None, None]
    h_g = [None, None]
    h_w = [None, None]
    h_i[0] = issue_idx(0)
    for g in range(_GNO):
        b = g & 1
        if h_w[b] is not None:
            for h in h_w[b]:
                h.wait()
        for h in h_i[b]:
            h.wait()
        h_g[b] = issue_gathers(g)
        if g >= 1:
            for h in h_g[1 - b]:
                h.wait()
            add_rows(1 - b)
            h_w[1 - b] = issue_writeback(g - 1)
            if g + 1 < _GNO:
                h_i[1 - b] = issue_idx(g + 1)
        else:
            h_i[1] = issue_idx(1)
    bl = (_GNO - 1) & 1
    for h in h_g[bl]:
        h.wait()
    add_rows(bl)
    h_w[bl] = issue_writeback(_GNO - 1)
    for hw in h_w:
        if hw is not None:
            for h in hw:
                h.wait()

    tbase = base0 + _GNO * _GS
    tsl = pl.ds(0, _GT)
    pltpu.sync_copy(snd_hbm.at[pl.ds(tbase, _GT)], si0.at[tsl])
    pltpu.sync_copy(rcv_hbm.at[pl.ds(tbase, _GT)], ri0.at[tsl])
    pltpu.async_copy(ps_hbm.at[si0.at[tsl]], pbuf.at[0, tsl], sem_g0).wait()
    pltpu.async_copy(pr_hbm.at[ri0.at[tsl]], rbuf.at[0, tsl], sem_g0).wait()

    def tbody(i, carry):
        for j in range(_D // 16):
            sl = pl.ds(j * 16, 16)
            pbuf[0, i, sl] = pbuf[0, i, sl] + rbuf[0, i, sl]
        return carry
    lax.fori_loop(0, _GT, tbody, 0)
    pltpu.sync_copy(pbuf.at[0, tsl], o_hbm.at[pl.ds(tbase, _GT)])


def _gather(ps, pr, senders, receivers):
    mesh = plsc.VectorSubcoreMesh(core_axis_name="c", subcore_axis_name="s")
    f = functools.partial(
        pl.kernel,
        out_type=jax.ShapeDtypeStruct((_E, _D), jnp.float32),
        mesh=mesh,
        scratch_types=[pltpu.VMEM((_GS,), jnp.int32),
                       pltpu.VMEM((_GS,), jnp.int32),
                       pltpu.VMEM((_GS,), jnp.int32),
                       pltpu.VMEM((_GS,), jnp.int32),
                       pltpu.VMEM((2, _GS, _D), jnp.float32),
                       pltpu.VMEM((2, _GS, _D), jnp.float32),
                       pltpu.SemaphoreType.DMA,
                       pltpu.SemaphoreType.DMA,
                       pltpu.SemaphoreType.DMA,
                       pltpu.SemaphoreType.DMA,
                       pltpu.SemaphoreType.DMA,
                       pltpu.SemaphoreType.DMA],
    )(_gather_body)
    return f(ps, pr, senders, receivers)


def _edge_body(e_ref, g_ref, we_ref, b1_ref, w2_ref, b2_ref, o_ref):
    x = jnp.dot(e_ref[...], we_ref[...], preferred_element_type=jnp.float32)
    h = jnp.maximum(x + g_ref[...] + b1_ref[...], 0.0)
    o_ref[...] = (jnp.dot(h, w2_ref[...], preferred_element_type=jnp.float32)
                  + b2_ref[...])


def _edge_mlp(edges, gsum, we, be1, we2, be2):
    eb = 4000
    blk = lambda i: (i, 0)
    wspec = pl.BlockSpec((_D, _D), lambda i: (0, 0))
    bspec = pl.BlockSpec((1, _D), lambda i: (0, 0))
    return pl.pallas_call(
        _edge_body,
        grid=(_E // eb,),
        in_specs=[pl.BlockSpec((eb, _D), blk),
                  pl.BlockSpec((eb, _D), blk),
                  wspec, bspec, wspec, bspec],
        out_specs=pl.BlockSpec((eb, _D), blk),
        out_shape=jax.ShapeDtypeStruct((_E, _D), jnp.float32),
    )(edges, gsum, we, be1.reshape(1, _D), we2, be2.reshape(1, _D))


def _scatter_body(ne_hbm, rcv_hbm, zeros_hbm, sums_hbm,
                  idx0, idx1, idxt, rows, acc_s,
                  sem_l0, sem_l1, sem_a0, sem_a1):
    cid = lax.axis_index("c")
    sid = lax.axis_index("s")
    wid = sid * 2 + cid
    srow = sid * _STRIPE
    idx_v = (idx0, idx1)
    sem_l = (sem_l0, sem_l1)
    sem_a = (sem_a0, sem_a1)
    pltpu.sync_copy(zeros_hbm.at[pl.ds(srow, _STRIPE)],
                    acc_s.at[pl.ds(srow, _STRIPE)])
    plsc.subcore_barrier()

    base0 = wid * _EPT

    def issue_loads(g):
        b = g & 1
        base = base0 + g * _C2
        return [pltpu.async_copy(ne_hbm.at[pl.ds(base, _C2)], rows.at[b],
                                 sem_l[b]),
                pltpu.async_copy(rcv_hbm.at[pl.ds(base, _C2)], idx_v[b],
                                 sem_l[b])]

    h_l = [None, None]
    h_a = [None, None]
    h_l[0] = issue_loads(0)
    for g in range(_NC2):
        b = g & 1
        for h in h_l[b]:
            h.wait()
        if g + 1 < _NC2:
            if h_a[1 - b] is not None:
                h_a[1 - b].wait()
            h_l[1 - b] = issue_loads(g + 1)
        h_a[b] = pltpu.async_copy(rows.at[b], acc_s.at[idx_v[b]], sem_a[b],
                                  add=True)
    for ha in h_a:
        if ha is not None:
            ha.wait()

    tbase = base0 + _NC2 * _C2
    pltpu.sync_copy(rcv_hbm.at[pl.ds(tbase, _TAIL)], idxt)
    pltpu.sync_copy(ne_hbm.at[pl.ds(tbase, _TAIL)], rows.at[0, pl.ds(0, _TAIL)])
    pltpu.sync_copy(rows.at[0, pl.ds(0, _TAIL)], acc_s.at[idxt], add=True)

    plsc.subcore_barrier()
    pltpu.sync_copy(acc_s.at[pl.ds(srow, _STRIPE)],
                    sums_hbm.at[cid, pl.ds(srow, _STRIPE)])


def _scatter(new_edges, receivers, zeros):
    mesh = plsc.VectorSubcoreMesh(core_axis_name="c", subcore_axis_name="s")
    f = functools.partial(
        pl.kernel,
        out_type=jax.ShapeDtypeStruct((2, _NP, _D), jnp.float32),
        mesh=mesh,
        scratch_types=[pltpu.VMEM((_C2,), jnp.int32),
                       pltpu.VMEM((_C2,), jnp.int32),
                       pltpu.VMEM((_TAIL,), jnp.int32),
                       pltpu.VMEM((2, _C2, _D), jnp.float32),
                       pltpu.VMEM_SHARED((_NP, _D), jnp.float32),
                       pltpu.SemaphoreType.DMA,
                       pltpu.SemaphoreType.DMA,
                       pltpu.SemaphoreType.DMA,
                       pltpu.SemaphoreType.DMA],
    )(_scatter_body)
    return f(new_edges, receivers, zeros)


def _counts_body(rcv_hbm, zeros_hbm, ones_hbm, cnts_hbm,
                 idx0, idx1, idxt, ones_v, acc_c,
                 sem_i0, sem_i1, sem_a0, sem_a1):
    cid = lax.axis_index("c")
    sid = lax.axis_index("s")
    wid = sid * 2 + cid
    srow = sid * _STRIPE
    idx_v = (idx0, idx1)
    sem_i = (sem_i0, sem_i1)
    sem_a = (sem_a0, sem_a1)
    pltpu.sync_copy(zeros_hbm.at[pl.ds(srow, _STRIPE)],
                    acc_c.at[pl.ds(srow, _STRIPE)])
    pltpu.sync_copy(ones_hbm, ones_v)
    plsc.subcore_barrier()

    base0 = wid * _EPT

    def issue_idx(g):
        b = g & 1
        base = base0 + g * _C2
        return [pltpu.async_copy(rcv_hbm.at[pl.ds(base, _C2)], idx_v[b],
                                 sem_i[b])]

    h_i = [None, None]
    h_a = [None, None]
    h_i[0] = issue_idx(0)
    for g in range(_NC2):
        b = g & 1
        for h in h_i[b]:
            h.wait()
        if g + 1 < _NC2:
            if h_a[1 - b] is not None:
                h_a[1 - b].wait()
            h_i[1 - b] = issue_idx(g + 1)
        h_a[b] = pltpu.async_copy(ones_v, acc_c.at[idx_v[b]], sem_a[b],
                                  add=True)
    for ha in h_a:
        if ha is not None:
            ha.wait()

    tbase = base0 + _NC2 * _C2
    pltpu.sync_copy(rcv_hbm.at[pl.ds(tbase, _TAIL)], idxt)
    pltpu.sync_copy(ones_v.at[pl.ds(0, _TAIL)], acc_c.at[idxt], add=True)

    plsc.subcore_barrier()
    pltpu.sync_copy(acc_c.at[pl.ds(srow, _STRIPE)],
                    cnts_hbm.at[cid, pl.ds(srow, _STRIPE)])


def _counts(receivers, zeros, ones):
    mesh = plsc.VectorSubcoreMesh(core_axis_name="c", subcore_axis_name="s")
    f = functools.partial(
        pl.kernel,
        out_type=jax.ShapeDtypeStruct((2, _NP, _D), jnp.float32),
        mesh=mesh,
        scratch_types=[pltpu.VMEM((_C2,), jnp.int32),
                       pltpu.VMEM((_C2,), jnp.int32),
                       pltpu.VMEM((_TAIL,), jnp.int32),
                       pltpu.VMEM((_C2, _D), jnp.float32),
                       pltpu.VMEM_SHARED((_NP, _D), jnp.float32),
                       pltpu.SemaphoreType.DMA,
                       pltpu.SemaphoreType.DMA,
                       pltpu.SemaphoreType.DMA,
                       pltpu.SemaphoreType.DMA],
    )(_counts_body)
    return f(receivers, zeros, ones)


def _node_body(s0_ref, s1_ref, c0_ref, c1_ref, pn_ref, w_ref, b1_ref,
               w2_ref, b2_ref, o_ref):
    sums = s0_ref[0] + s1_ref[0]
    cnt = jnp.max(c0_ref[0] + c1_ref[0], axis=1, keepdims=True)
    agg = sums / jnp.maximum(cnt, 1.0)
    hn = jnp.maximum(
        pn_ref[...]
        + jnp.dot(agg, w_ref[...], preferred_element_type=jnp.float32)
        + b1_ref[...], 0.0)
    o_ref[...] = (jnp.dot(hn, w2_ref[...], preferred_element_type=jnp.float32)
                  + b2_ref[...])


def _node_mlp(sums, cnts, pn, wn1b, bn1, wn2, bn2):
    nb = 1000
    blk = lambda i: (i, 0)
    j0 = pl.BlockSpec((1, nb, _D), lambda i: (0, i, 0))
    j1 = pl.BlockSpec((1, nb, _D), lambda i: (1, i, 0))
    wspec = pl.BlockSpec((_D, _D), lambda i: (0, 0))
    bspec = pl.BlockSpec((1, _D), lambda i: (0, 0))
    return pl.pallas_call(
        _node_body,
        grid=(_N // nb,),
        in_specs=[j0, j1, j0, j1,
                  pl.BlockSpec((nb, _D), blk),
                  wspec, bspec, wspec, bspec],
        out_specs=pl.BlockSpec((nb, _D), blk),
        out_shape=jax.ShapeDtypeStruct((_N, _D), jnp.float32),
    )(sums, sums, cnts, cnts, pn, wn1b, bn1.reshape(1, _D), wn2,
      bn2.reshape(1, _D))


def kernel(nodes, edges, senders, receivers,
           We1, be1, We2, be2, Wn1, bn1, Wn2, bn2):
    we = We1[0:_D]
    ws = We1[_D:2 * _D]
    wr = We1[2 * _D:3 * _D]
    wn1a = Wn1[0:_D]
    wn1b = Wn1[_D:2 * _D]

    ps, pr, pn = _proj(nodes, jnp.concatenate([ws, wr, wn1a], axis=1))

    gsum = _gather(ps, pr, senders, receivers)
    new_edges = _edge_mlp(edges, gsum, we, be1, We2, be2)

    zeros = jnp.zeros((_NP, _D), jnp.float32)
    ones = jnp.ones((_C2, _D), jnp.float32)
    cnts = _counts(receivers, zeros, ones)
    sums = _scatter(new_edges, receivers, zeros)

    new_nodes = _node_mlp(sums, cnts, pn, wn1b, bn1, Wn2, bn2)
    return (new_nodes, new_edges)


# speedup vs baseline: 1.3802x; 1.0191x over previous
import functools

import jax
import jax.numpy as jnp
from jax import lax
from jax.experimental import pallas as pl
from jax.experimental.pallas import tpu as pltpu
from jax.experimental.pallas import tpu_sc as plsc

_N = 10000
_E = 320000
_D = 128
_H = _D // 2

_NW = 32
_EPT = _E // _NW
_S = 400
_SUB = 80
_K = _S // _SUB
_NO = _EPT // _S
_NP = 10240
_STRIPE = _NP // 16
_C2 = 128
_NC2 = _EPT // _C2
_TAIL = _EPT - _NC2 * _C2


def _proj_body(n_ref, w_ref, ps_ref, pr_ref, pn_ref):
    o = jnp.dot(n_ref[...], w_ref[...], preferred_element_type=jnp.float32)
    ps_ref[...] = o[:, 0:_D]
    pr_ref[...] = o[:, _D:2 * _D]
    pn_ref[...] = o[:, 2 * _D:3 * _D]


def _proj(nodes, wcat):
    nb = 2000
    blk = lambda i: (i, 0)
    return pl.pallas_call(
        _proj_body,
        grid=(_N // nb,),
        in_specs=[pl.BlockSpec((nb, _D), blk),
                  pl.BlockSpec((_D, 3 * _D), lambda i: (0, 0))],
        out_specs=[pl.BlockSpec((nb, _D), blk),
                   pl.BlockSpec((nb, _D), blk),
                   pl.BlockSpec((nb, _D), blk)],
        out_shape=[jax.ShapeDtypeStruct((_N, _D), jnp.float32),
                   jax.ShapeDtypeStruct((_N, _D), jnp.float32),
                   jax.ShapeDtypeStruct((_N, _D), jnp.float32)],
    )(nodes, wcat)


_GS = 120
_GNO = _EPT // _GS
_GT = _EPT - _GNO * _GS


def _gather_body(ps_hbm, pr_hbm, snd_hbm, rcv_hbm, o_hbm,
                 si0, si1, ri0, ri1, pbuf, rbuf,
                 sem_i0, sem_i1, sem_g0, sem_g1, sem_w0, sem_w1):
    wid = lax.axis_index("s") * 2 + lax.axis_index("c")
    base0 = wid * _EPT
    si = (si0, si1)
    ri = (ri0, ri1)
    sem_i = (sem_i0, sem_i1)
    sem_g = (sem_g0, sem_g1)
    sem_w = (sem_w0, sem_w1)

    def issue_idx(g):
        b = g & 1
        base = base0 + g * _GS
        return [pltpu.async_copy(snd_hbm.at[pl.ds(base, _GS)], si[b],
                                 sem_i[b]),
                pltpu.async_copy(rcv_hbm.at[pl.ds(base, _GS)], ri[b],
                                 sem_i[b])]

    def issue_gathers(g):
        b = g & 1
        return [pltpu.async_copy(ps_hbm.at[si[b]], pbuf.at[b], sem_g[b]),
                pltpu.async_copy(pr_hbm.at[ri[b]], rbuf.at[b], sem_g[b])]

    def add_rows(b):
        def body(i, carry):
            for j in range(_D // 16):
                sl = pl.ds(j * 16, 16)
                pbuf[b, i, sl] = pbuf[b, i, sl] + rbuf[b, i, sl]
            return carry
        lax.fori_loop(0, _GS, body, 0)

    def issue_writeback(g):
        b = g & 1
        base = base0 + g * _GS
        return [pltpu.async_copy(pbuf.at[b], o_hbm.at[pl.ds(base, _GS)],
                                 sem_w[b])]

    h_i = [None, None]
    h_g = [None, None]
    h_w = [None, None]
    h_i[0] = issue_idx(0)
    for g in range(_GNO):
        b = g & 1
        if h_w[b] is not None:
            for h in h_w[b]:
                h.wait()
        for h in h_i[b]:
            h.wait()
        h_g[b] = issue_gathers(g)
        if g >= 1:
            for h in h_g[1 - b]:
                h.wait()
            add_rows(1 - b)
            h_w[1 - b] = issue_writeback(g - 1)
            if g + 1 < _GNO:
                h_i[1 - b] = issue_idx(g + 1)
        else:
            h_i[1] = issue_idx(1)
    bl = (_GNO - 1) & 1
    for h in h_g[bl]:
        h.wait()
    add_rows(bl)
    h_w[bl] = issue_writeback(_GNO - 1)
    for hw in h_w:
        if hw is not None:
            for h in hw:
                h.wait()

    tbase = base0 + _GNO * _GS
    tsl = pl.ds(0, _GT)
    pltpu.sync_copy(snd_hbm.at[pl.ds(tbase, _GT)], si0.at[tsl])
    pltpu.sync_copy(rcv_hbm.at[pl.ds(tbase, _GT)], ri0.at[tsl])
    pltpu.async_copy(ps_hbm.at[si0.at[tsl]], pbuf.at[0, tsl], sem_g0).wait()
    pltpu.async_copy(pr_hbm.at[ri0.at[tsl]], rbuf.at[0, tsl], sem_g0).wait()

    def tbody(i, carry):
        for j in range(_D // 16):
            sl = pl.ds(j * 16, 16)
            pbuf[0, i, sl] = pbuf[0, i, sl] + rbuf[0, i, sl]
        return carry
    lax.fori_loop(0, _GT, tbody, 0)
    pltpu.sync_copy(pbuf.at[0, tsl], o_hbm.at[pl.ds(tbase, _GT)])


def _gather(ps, pr, senders, receivers):
    mesh = plsc.VectorSubcoreMesh(core_axis_name="c", subcore_axis_name="s")
    f = functools.partial(
        pl.kernel,
        out_type=jax.ShapeDtypeStruct((_E, _D), jnp.float32),
        mesh=mesh,
        scratch_types=[pltpu.VMEM((_GS,), jnp.int32),
                       pltpu.VMEM((_GS,), jnp.int32),
                       pltpu.VMEM((_GS,), jnp.int32),
                       pltpu.VMEM((_GS,), jnp.int32),
                       pltpu.VMEM((2, _GS, _D), jnp.float32),
                       pltpu.VMEM((2, _GS, _D), jnp.float32),
                       pltpu.SemaphoreType.DMA,
                       pltpu.SemaphoreType.DMA,
                       pltpu.SemaphoreType.DMA,
                       pltpu.SemaphoreType.DMA,
                       pltpu.SemaphoreType.DMA,
                       pltpu.SemaphoreType.DMA],
    )(_gather_body)
    return f(ps, pr, senders, receivers)


def _edge_body(e_ref, g_ref, we_ref, b1_ref, w2_ref, b2_ref, o_ref):
    x = jnp.dot(e_ref[...], we_ref[...], preferred_element_type=jnp.float32)
    h = jnp.maximum(x + g_ref[...] + b1_ref[...], 0.0)
    o_ref[...] = (jnp.dot(h, w2_ref[...], preferred_element_type=jnp.float32)
                  + b2_ref[...])


def _edge_mlp(edges, gsum, we, be1, we2, be2):
    eb = 8000
    blk = lambda i: (i, 0)
    wspec = pl.BlockSpec((_D, _D), lambda i: (0, 0))
    bspec = pl.BlockSpec((1, _D), lambda i: (0, 0))
    return pl.pallas_call(
        _edge_body,
        grid=(_E // eb,),
        in_specs=[pl.BlockSpec((eb, _D), blk),
                  pl.BlockSpec((eb, _D), blk),
                  wspec, bspec, wspec, bspec],
        out_specs=pl.BlockSpec((eb, _D), blk),
        out_shape=jax.ShapeDtypeStruct((_E, _D), jnp.float32),
    )(edges, gsum, we, be1.reshape(1, _D), we2, be2.reshape(1, _D))


def _scatter_body(ne_hbm, rcv_hbm, zeros_hbm, sums_hbm,
                  idx0, idx1, idxt, rows, acc_s,
                  sem_l0, sem_l1, sem_a0, sem_a1):
    cid = lax.axis_index("c")
    sid = lax.axis_index("s")
    wid = sid * 2 + cid
    srow = sid * _STRIPE
    idx_v = (idx0, idx1)
    sem_l = (sem_l0, sem_l1)
    sem_a = (sem_a0, sem_a1)
    pltpu.sync_copy(zeros_hbm.at[pl.ds(srow, _STRIPE)],
                    acc_s.at[pl.ds(srow, _STRIPE)])
    plsc.subcore_barrier()

    base0 = wid * _EPT

    def issue_loads(g):
        b = g & 1
        base = base0 + g * _C2
        return [pltpu.async_copy(ne_hbm.at[pl.ds(base, _C2)], rows.at[b],
                                 sem_l[b]),
                pltpu.async_copy(rcv_hbm.at[pl.ds(base, _C2)], idx_v[b],
                                 sem_l[b])]

    h_l = [None, None]
    h_a = [None, None]
    h_l[0] = issue_loads(0)
    for g in range(_NC2):
        b = g & 1
        for h in h_l[b]:
            h.wait()
        if g + 1 < _NC2:
            if h_a[1 - b] is not None:
                h_a[1 - b].wait()
            h_l[1 - b] = issue_loads(g + 1)
        h_a[b] = pltpu.async_copy(rows.at[b], acc_s.at[idx_v[b]], sem_a[b],
                                  add=True)
    for ha in h_a:
        if ha is not None:
            ha.wait()

    tbase = base0 + _NC2 * _C2
    pltpu.sync_copy(rcv_hbm.at[pl.ds(tbase, _TAIL)], idxt)
    pltpu.sync_copy(ne_hbm.at[pl.ds(tbase, _TAIL)], rows.at[0, pl.ds(0, _TAIL)])
    pltpu.sync_copy(rows.at[0, pl.ds(0, _TAIL)], acc_s.at[idxt], add=True)

    plsc.subcore_barrier()
    pltpu.sync_copy(acc_s.at[pl.ds(srow, _STRIPE)],
                    sums_hbm.at[cid, pl.ds(srow, _STRIPE)])


def _scatter(new_edges, receivers, zeros):
    mesh = plsc.VectorSubcoreMesh(core_axis_name="c", subcore_axis_name="s")
    f = functools.partial(
        pl.kernel,
        out_type=jax.ShapeDtypeStruct((2, _NP, _D), jnp.float32),
        mesh=mesh,
        scratch_types=[pltpu.VMEM((_C2,), jnp.int32),
                       pltpu.VMEM((_C2,), jnp.int32),
                       pltpu.VMEM((_TAIL,), jnp.int32),
                       pltpu.VMEM((2, _C2, _D), jnp.float32),
                       pltpu.VMEM_SHARED((_NP, _D), jnp.float32),
                       pltpu.SemaphoreType.DMA,
                       pltpu.SemaphoreType.DMA,
                       pltpu.SemaphoreType.DMA,
                       pltpu.SemaphoreType.DMA],
    )(_scatter_body)
    return f(new_edges, receivers, zeros)


def _counts_body(rcv_hbm, zeros_hbm, ones_hbm, cnts_hbm,
                 idx0, idx1, idxt, ones_v, acc_c,
                 sem_i0, sem_i1, sem_a0, sem_a1):
    cid = lax.axis_index("c")
    sid = lax.axis_index("s")
    wid = sid * 2 + cid
    srow = sid * _STRIPE
    idx_v = (idx0, idx1)
    sem_i = (sem_i0, sem_i1)
    sem_a = (sem_a0, sem_a1)
    pltpu.sync_copy(zeros_hbm.at[pl.ds(srow, _STRIPE)],
                    acc_c.at[pl.ds(srow, _STRIPE)])
    pltpu.sync_copy(ones_hbm, ones_v)
    plsc.subcore_barrier()

    base0 = wid * _EPT

    def issue_idx(g):
        b = g & 1
        base = base0 + g * _C2
        return [pltpu.async_copy(rcv_hbm.at[pl.ds(base, _C2)], idx_v[b],
                                 sem_i[b])]

    h_i = [None, None]
    h_a = [None, None]
    h_i[0] = issue_idx(0)
    for g in range(_NC2):
        b = g & 1
        for h in h_i[b]:
            h.wait()
        if g + 1 < _NC2:
            if h_a[1 - b] is not None:
                h_a[1 - b].wait()
            h_i[1 - b] = issue_idx(g + 1)
        h_a[b] = pltpu.async_copy(ones_v, acc_c.at[idx_v[b]], sem_a[b],
                                  add=True)
    for ha in h_a:
        if ha is not None:
            ha.wait()

    tbase = base0 + _NC2 * _C2
    pltpu.sync_copy(rcv_hbm.at[pl.ds(tbase, _TAIL)], idxt)
    pltpu.sync_copy(ones_v.at[pl.ds(0, _TAIL)], acc_c.at[idxt], add=True)

    plsc.subcore_barrier()
    pltpu.sync_copy(acc_c.at[pl.ds(srow, _STRIPE)],
                    cnts_hbm.at[cid, pl.ds(srow, _STRIPE)])


def _counts(receivers, zeros, ones):
    mesh = plsc.VectorSubcoreMesh(core_axis_name="c", subcore_axis_name="s")
    f = functools.partial(
        pl.kernel,
        out_type=jax.ShapeDtypeStruct((2, _NP, _D), jnp.float32),
        mesh=mesh,
        scratch_types=[pltpu.VMEM((_C2,), jnp.int32),
                       pltpu.VMEM((_C2,), jnp.int32),
                       pltpu.VMEM((_TAIL,), jnp.int32),
                       pltpu.VMEM((_C2, _D), jnp.float32),
                       pltpu.VMEM_SHARED((_NP, _D), jnp.float32),
                       pltpu.SemaphoreType.DMA,
                       pltpu.SemaphoreType.DMA,
                       pltpu.SemaphoreType.DMA,
                       pltpu.SemaphoreType.DMA],
    )(_counts_body)
    return f(receivers, zeros, ones)


def _node_body(s0_ref, s1_ref, c0_ref, c1_ref, pn_ref, w_ref, b1_ref,
               w2_ref, b2_ref, o_ref):
    sums = s0_ref[0] + s1_ref[0]
    cnt = jnp.max(c0_ref[0] + c1_ref[0], axis=1, keepdims=True)
    agg = sums / jnp.maximum(cnt, 1.0)
    hn = jnp.maximum(
        pn_ref[...]
        + jnp.dot(agg, w_ref[...], preferred_element_type=jnp.float32)
        + b1_ref[...], 0.0)
    o_ref[...] = (jnp.dot(hn, w2_ref[...], preferred_element_type=jnp.float32)
                  + b2_ref[...])


def _node_mlp(sums, cnts, pn, wn1b, bn1, wn2, bn2):
    nb = 1000
    blk = lambda i: (i, 0)
    j0 = pl.BlockSpec((1, nb, _D), lambda i: (0, i, 0))
    j1 = pl.BlockSpec((1, nb, _D), lambda i: (1, i, 0))
    wspec = pl.BlockSpec((_D, _D), lambda i: (0, 0))
    bspec = pl.BlockSpec((1, _D), lambda i: (0, 0))
    return pl.pallas_call(
        _node_body,
        grid=(_N // nb,),
        in_specs=[j0, j1, j0, j1,
                  pl.BlockSpec((nb, _D), blk),
                  wspec, bspec, wspec, bspec],
        out_specs=pl.BlockSpec((nb, _D), blk),
        out_shape=jax.ShapeDtypeStruct((_N, _D), jnp.float32),
    )(sums, sums, cnts, cnts, pn, wn1b, bn1.reshape(1, _D), wn2,
      bn2.reshape(1, _D))


def kernel(nodes, edges, senders, receivers,
           We1, be1, We2, be2, Wn1, bn1, Wn2, bn2):
    we = We1[0:_D]
    ws = We1[_D:2 * _D]
    wr = We1[2 * _D:3 * _D]
    wn1a = Wn1[0:_D]
    wn1b = Wn1[_D:2 * _D]

    ps, pr, pn = _proj(nodes, jnp.concatenate([ws, wr, wn1a], axis=1))

    gsum = _gather(ps, pr, senders, receivers)
    new_edges = _edge_mlp(edges, gsum, we, be1, We2, be2)

    zeros = jnp.zeros((_NP, _D), jnp.float32)
    ones = jnp.ones((_C2, _D), jnp.float32)
    cnts = _counts(receivers, zeros, ones)
    sums = _scatter(new_edges, receivers, zeros)

    new_nodes = _node_mlp(sums, cnts, pn, wn1b, bn1, Wn2, bn2)
    return (new_nodes, new_edges)


# speedup vs baseline: 1.3894x; 1.0066x over previous
import functools

import jax
import jax.numpy as jnp
from jax import lax
from jax.experimental import pallas as pl
from jax.experimental.pallas import tpu as pltpu
from jax.experimental.pallas import tpu_sc as plsc

_N = 10000
_E = 320000
_D = 128
_H = _D // 2

_NW = 32
_EPT = _E // _NW
_S = 400
_SUB = 80
_K = _S // _SUB
_NO = _EPT // _S
_NP = 10240
_STRIPE = _NP // 16
_C2 = 128
_NC2 = _EPT // _C2
_TAIL = _EPT - _NC2 * _C2


def _proj_body(n_ref, w_ref, ps_ref, pr_ref, pn_ref):
    o = jnp.dot(n_ref[...], w_ref[...], preferred_element_type=jnp.float32)
    ps_ref[...] = o[:, 0:_D]
    pr_ref[...] = o[:, _D:2 * _D]
    pn_ref[...] = o[:, 2 * _D:3 * _D]


def _proj(nodes, wcat):
    nb = 2000
    blk = lambda i: (i, 0)
    return pl.pallas_call(
        _proj_body,
        grid=(_N // nb,),
        in_specs=[pl.BlockSpec((nb, _D), blk),
                  pl.BlockSpec((_D, 3 * _D), lambda i: (0, 0))],
        out_specs=[pl.BlockSpec((nb, _D), blk),
                   pl.BlockSpec((nb, _D), blk),
                   pl.BlockSpec((nb, _D), blk)],
        out_shape=[jax.ShapeDtypeStruct((_N, _D), jnp.float32),
                   jax.ShapeDtypeStruct((_N, _D), jnp.float32),
                   jax.ShapeDtypeStruct((_N, _D), jnp.float32)],
    )(nodes, wcat)


_GS = 120
_GNO = _EPT // _GS
_GT = _EPT - _GNO * _GS


def _gather_body(ps_hbm, pr_hbm, snd_hbm, rcv_hbm, o_hbm,
                 si0, si1, ri0, ri1, pbuf, rbuf,
                 sem_i0, sem_i1, sem_g0, sem_g1, sem_w0, sem_w1):
    wid = lax.axis_index("s") * 2 + lax.axis_index("c")
    base0 = wid * _EPT
    si = (si0, si1)
    ri = (ri0, ri1)
    sem_i = (sem_i0, sem_i1)
    sem_g = (sem_g0, sem_g1)
    sem_w = (sem_w0, sem_w1)

    def issue_idx(g):
        b = g & 1
        base = base0 + g * _GS
        return [pltpu.async_copy(snd_hbm.at[pl.ds(base, _GS)], si[b],
                                 sem_i[b]),
                pltpu.async_copy(rcv_hbm.at[pl.ds(base, _GS)], ri[b],
                                 sem_i[b])]

    def issue_gathers(g):
        b = g & 1
        return [pltpu.async_copy(ps_hbm.at[si[b]], pbuf.at[b], sem_g[b]),
                pltpu.async_copy(pr_hbm.at[ri[b]], rbuf.at[b], sem_g[b])]

    def add_rows(b):
        def body(i, carry):
            for j in range(_D // 16):
                sl = pl.ds(j * 16, 16)
                pbuf[b, i, sl] = pbuf[b, i, sl] + rbuf[b, i, sl]
            return carry
        lax.fori_loop(0, _GS, body, 0)

    def issue_writeback(g):
        b = g & 1
        base = base0 + g * _GS
        return [pltpu.async_copy(pbuf.at[b], o_hbm.at[pl.ds(base, _GS)],
                                 sem_w[b])]

    h_i = [None, None]
    h_g = [None, None]
    h_w = [None, None]
    h_i[0] = issue_idx(0)
    for g in range(_GNO):
        b = g & 1
        if h_w[b] is not None:
            for h in h_w[b]:
                h.wait()
        for h in h_i[b]:
            h.wait()
        h_g[b] = issue_gathers(g)
        if g >= 1:
            for h in h_g[1 - b]:
                h.wait()
            add_rows(1 - b)
            h_w[1 - b] = issue_writeback(g - 1)
            if g + 1 < _GNO:
                h_i[1 - b] = issue_idx(g + 1)
        else:
            h_i[1] = issue_idx(1)
    bl = (_GNO - 1) & 1
    for h in h_g[bl]:
        h.wait()
    add_rows(bl)
    h_w[bl] = issue_writeback(_GNO - 1)
    for hw in h_w:
        if hw is not None:
            for h in hw:
                h.wait()

    tbase = base0 + _GNO * _GS
    tsl = pl.ds(0, _GT)
    pltpu.sync_copy(snd_hbm.at[pl.ds(tbase, _GT)], si0.at[tsl])
    pltpu.sync_copy(rcv_hbm.at[pl.ds(tbase, _GT)], ri0.at[tsl])
    pltpu.async_copy(ps_hbm.at[si0.at[tsl]], pbuf.at[0, tsl], sem_g0).wait()
    pltpu.async_copy(pr_hbm.at[ri0.at[tsl]], rbuf.at[0, tsl], sem_g0).wait()

    def tbody(i, carry):
        for j in range(_D // 16):
            sl = pl.ds(j * 16, 16)
            pbuf[0, i, sl] = pbuf[0, i, sl] + rbuf[0, i, sl]
        return carry
    lax.fori_loop(0, _GT, tbody, 0)
    pltpu.sync_copy(pbuf.at[0, tsl], o_hbm.at[pl.ds(tbase, _GT)])


def _gather(ps, pr, senders, receivers):
    mesh = plsc.VectorSubcoreMesh(core_axis_name="c", subcore_axis_name="s")
    f = functools.partial(
        pl.kernel,
        out_type=jax.ShapeDtypeStruct((_E, _D), jnp.float32),
        mesh=mesh,
        scratch_types=[pltpu.VMEM((_GS,), jnp.int32),
                       pltpu.VMEM((_GS,), jnp.int32),
                       pltpu.VMEM((_GS,), jnp.int32),
                       pltpu.VMEM((_GS,), jnp.int32),
                       pltpu.VMEM((2, _GS, _D), jnp.float32),
                       pltpu.VMEM((2, _GS, _D), jnp.float32),
                       pltpu.SemaphoreType.DMA,
                       pltpu.SemaphoreType.DMA,
                       pltpu.SemaphoreType.DMA,
                       pltpu.SemaphoreType.DMA,
                       pltpu.SemaphoreType.DMA,
                       pltpu.SemaphoreType.DMA],
    )(_gather_body)
    return f(ps, pr, senders, receivers)


def _edge_body(e_ref, g_ref, we_ref, b1_ref, w2_ref, b2_ref, o_ref):
    x = jnp.dot(e_ref[...], we_ref[...], preferred_element_type=jnp.float32)
    h = jnp.maximum(x + g_ref[...] + b1_ref[...], 0.0)
    o_ref[...] = (jnp.dot(h, w2_ref[...], preferred_element_type=jnp.float32)
                  + b2_ref[...])


def _edge_mlp(edges, gsum, we, be1, we2, be2):
    eb = 16000
    blk = lambda i: (i, 0)
    wspec = pl.BlockSpec((_D, _D), lambda i: (0, 0))
    bspec = pl.BlockSpec((1, _D), lambda i: (0, 0))
    return pl.pallas_call(
        _edge_body,
        grid=(_E // eb,),
        in_specs=[pl.BlockSpec((eb, _D), blk),
                  pl.BlockSpec((eb, _D), blk),
                  wspec, bspec, wspec, bspec],
        out_specs=pl.BlockSpec((eb, _D), blk),
        out_shape=jax.ShapeDtypeStruct((_E, _D), jnp.float32),
    )(edges, gsum, we, be1.reshape(1, _D), we2, be2.reshape(1, _D))


def _scatter_body(ne_hbm, rcv_hbm, zeros_hbm, sums_hbm,
                  idx0, idx1, idxt, rows, acc_s,
                  sem_l0, sem_l1, sem_a0, sem_a1):
    cid = lax.axis_index("c")
    sid = lax.axis_index("s")
    wid = sid * 2 + cid
    srow = sid * _STRIPE
    idx_v = (idx0, idx1)
    sem_l = (sem_l0, sem_l1)
    sem_a = (sem_a0, sem_a1)
    pltpu.sync_copy(zeros_hbm.at[pl.ds(srow, _STRIPE)],
                    acc_s.at[pl.ds(srow, _STRIPE)])
    plsc.subcore_barrier()

    base0 = wid * _EPT

    def issue_loads(g):
        b = g & 1
        base = base0 + g * _C2
        return [pltpu.async_copy(ne_hbm.at[pl.ds(base, _C2)], rows.at[b],
                                 sem_l[b]),
                pltpu.async_copy(rcv_hbm.at[pl.ds(base, _C2)], idx_v[b],
                                 sem_l[b])]

    h_l = [None, None]
    h_a = [None, None]
    h_l[0] = issue_loads(0)
    for g in range(_NC2):
        b = g & 1
        for h in h_l[b]:
            h.wait()
        if g + 1 < _NC2:
            if h_a[1 - b] is not None:
                h_a[1 - b].wait()
            h_l[1 - b] = issue_loads(g + 1)
        h_a[b] = pltpu.async_copy(rows.at[b], acc_s.at[idx_v[b]], sem_a[b],
                                  add=True)
    for ha in h_a:
        if ha is not None:
            ha.wait()

    tbase = base0 + _NC2 * _C2
    pltpu.sync_copy(rcv_hbm.at[pl.ds(tbase, _TAIL)], idxt)
    pltpu.sync_copy(ne_hbm.at[pl.ds(tbase, _TAIL)], rows.at[0, pl.ds(0, _TAIL)])
    pltpu.sync_copy(rows.at[0, pl.ds(0, _TAIL)], acc_s.at[idxt], add=True)

    plsc.subcore_barrier()
    pltpu.sync_copy(acc_s.at[pl.ds(srow, _STRIPE)],
                    sums_hbm.at[cid, pl.ds(srow, _STRIPE)])


def _scatter(new_edges, receivers, zeros):
    mesh = plsc.VectorSubcoreMesh(core_axis_name="c", subcore_axis_name="s")
    f = functools.partial(
        pl.kernel,
        out_type=jax.ShapeDtypeStruct((2, _NP, _D), jnp.float32),
        mesh=mesh,
        scratch_types=[pltpu.VMEM((_C2,), jnp.int32),
                       pltpu.VMEM((_C2,), jnp.int32),
                       pltpu.VMEM((_TAIL,), jnp.int32),
                       pltpu.VMEM((2, _C2, _D), jnp.float32),
                       pltpu.VMEM_SHARED((_NP, _D), jnp.float32),
                       pltpu.SemaphoreType.DMA,
                       pltpu.SemaphoreType.DMA,
                       pltpu.SemaphoreType.DMA,
                       pltpu.SemaphoreType.DMA],
    )(_scatter_body)
    return f(new_edges, receivers, zeros)


def _counts_body(rcv_hbm, zeros_hbm, ones_hbm, cnts_hbm,
                 idx0, idx1, idxt, ones_v, acc_c,
                 sem_i0, sem_i1, sem_a0, sem_a1):
    cid = lax.axis_index("c")
    sid = lax.axis_index("s")
    wid = sid * 2 + cid
    srow = sid * _STRIPE
    idx_v = (idx0, idx1)
    sem_i = (sem_i0, sem_i1)
    sem_a = (sem_a0, sem_a1)
    pltpu.sync_copy(zeros_hbm.at[pl.ds(srow, _STRIPE)],
                    acc_c.at[pl.ds(srow, _STRIPE)])
    pltpu.sync_copy(ones_hbm, ones_v)
    plsc.subcore_barrier()

    base0 = wid * _EPT

    def issue_idx(g):
        b = g & 1
        base = base0 + g * _C2
        return [pltpu.async_copy(rcv_hbm.at[pl.ds(base, _C2)], idx_v[b],
                                 sem_i[b])]

    h_i = [None, None]
    h_a = [None, None]
    h_i[0] = issue_idx(0)
    for g in range(_NC2):
        b = g & 1
        for h in h_i[b]:
            h.wait()
        if g + 1 < _NC2:
            if h_a[1 - b] is not None:
                h_a[1 - b].wait()
            h_i[1 - b] = issue_idx(g + 1)
        h_a[b] = pltpu.async_copy(ones_v, acc_c.at[idx_v[b]], sem_a[b],
                                  add=True)
    for ha in h_a:
        if ha is not None:
            ha.wait()

    tbase = base0 + _NC2 * _C2
    pltpu.sync_copy(rcv_hbm.at[pl.ds(tbase, _TAIL)], idxt)
    pltpu.sync_copy(ones_v.at[pl.ds(0, _TAIL)], acc_c.at[idxt], add=True)

    plsc.subcore_barrier()
    pltpu.sync_copy(acc_c.at[pl.ds(srow, _STRIPE)],
                    cnts_hbm.at[cid, pl.ds(srow, _STRIPE)])


def _counts(receivers, zeros, ones):
    mesh = plsc.VectorSubcoreMesh(core_axis_name="c", subcore_axis_name="s")
    f = functools.partial(
        pl.kernel,
        out_type=jax.ShapeDtypeStruct((2, _NP, _D), jnp.float32),
        mesh=mesh,
        scratch_types=[pltpu.VMEM((_C2,), jnp.int32),
                       pltpu.VMEM((_C2,), jnp.int32),
                       pltpu.VMEM((_TAIL,), jnp.int32),
                       pltpu.VMEM((_C2, _D), jnp.float32),
                       pltpu.VMEM_SHARED((_NP, _D), jnp.float32),
                       pltpu.SemaphoreType.DMA,
                       pltpu.SemaphoreType.DMA,
                       pltpu.SemaphoreType.DMA,
                       pltpu.SemaphoreType.DMA],
    )(_counts_body)
    return f(receivers, zeros, ones)


def _node_body(s0_ref, s1_ref, c0_ref, c1_ref, pn_ref, w_ref, b1_ref,
               w2_ref, b2_ref, o_ref):
    sums = s0_ref[0] + s1_ref[0]
    cnt = jnp.max(c0_ref[0] + c1_ref[0], axis=1, keepdims=True)
    agg = sums / jnp.maximum(cnt, 1.0)
    hn = jnp.maximum(
        pn_ref[...]
        + jnp.dot(agg, w_ref[...], preferred_element_type=jnp.float32)
        + b1_ref[...], 0.0)
    o_ref[...] = (jnp.dot(hn, w2_ref[...], preferred_element_type=jnp.float32)
                  + b2_ref[...])


def _node_mlp(sums, cnts, pn, wn1b, bn1, wn2, bn2):
    nb = 1000
    blk = lambda i: (i, 0)
    j0 = pl.BlockSpec((1, nb, _D), lambda i: (0, i, 0))
    j1 = pl.BlockSpec((1, nb, _D), lambda i: (1, i, 0))
    wspec = pl.BlockSpec((_D, _D), lambda i: (0, 0))
    bspec = pl.BlockSpec((1, _D), lambda i: (0, 0))
    return pl.pallas_call(
        _node_body,
        grid=(_N // nb,),
        in_specs=[j0, j1, j0, j1,
                  pl.BlockSpec((nb, _D), blk),
                  wspec, bspec, wspec, bspec],
        out_specs=pl.BlockSpec((nb, _D), blk),
        out_shape=jax.ShapeDtypeStruct((_N, _D), jnp.float32),
    )(sums, sums, cnts, cnts, pn, wn1b, bn1.reshape(1, _D), wn2,
      bn2.reshape(1, _D))


def kernel(nodes, edges, senders, receivers,
           We1, be1, We2, be2, Wn1, bn1, Wn2, bn2):
    we = We1[0:_D]
    ws = We1[_D:2 * _D]
    wr = We1[2 * _D:3 * _D]
    wn1a = Wn1[0:_D]
    wn1b = Wn1[_D:2 * _D]

    ps, pr, pn = _proj(nodes, jnp.concatenate([ws, wr, wn1a], axis=1))

    gsum = _gather(ps, pr, senders, receivers)
    new_edges = _edge_mlp(edges, gsum, we, be1, We2, be2)

    zeros = jnp.zeros((_NP, _D), jnp.float32)
    ones = jnp.ones((_C2, _D), jnp.float32)
    cnts = _counts(receivers, zeros, ones)
    sums = _scatter(new_edges, receivers, zeros)

    new_nodes = _node_mlp(sums, cnts, pn, wn1b, bn1, Wn2, bn2)
    return (new_nodes, new_edges)
